# baseline (speedup 1.0000x reference)
import functools

import jax
import jax.numpy as jnp
from jax.experimental import pallas as pl
from jax.experimental.pallas import tpu as pltpu
from jax.experimental.pallas import tpu_sc as plsc

N = 10000
E = 160000
HID = 256
ZD = 128
DEC = HID + ZD

_NS = 16
_L = 16
_K = 80
_NPAD = 10240
_STRIPE = _NPAD // _NS


def _mm_body(x_ref, w_ref, b_ref, o_ref, *, relu):
    acc = jnp.dot(x_ref[...], w_ref[...], preferred_element_type=jnp.float32)
    acc = acc + b_ref[0, :][None, :]
    if relu:
        acc = jnp.maximum(acc, 0.0)
    o_ref[...] = acc


def _mm(x, w, b, relu=False):
    m, k = x.shape
    n = w.shape[1]
    bm = 640 if m == E else 400
    b8 = jnp.broadcast_to(b[None, :], (8, n))
    return pl.pallas_call(
        functools.partial(_mm_body, relu=relu),
        grid=(m // bm,),
        in_specs=[
            pl.BlockSpec((bm, k), lambda i: (i, 0)),
            pl.BlockSpec((k, n), lambda i: (0, 0)),
            pl.BlockSpec((8, n), lambda i: (0, 0)),
        ],
        out_specs=pl.BlockSpec((bm, n), lambda i: (i, 0)),
        out_shape=jax.ShapeDtypeStruct((m, n), jnp.float32),
    )(x, w, b8)


def _outer_body(ew_ref, wb_ref, o_ref):
    o_ref[...] = ew_ref[...] * wb_ref[0, :][None, :] + wb_ref[1, :][None, :]


def _outer(e_w, w_col, b):
    h = w_col.shape[0]
    wb = jnp.concatenate([w_col[None, :], b[None, :]], axis=0)
    wb8 = jnp.concatenate([wb, jnp.zeros((6, h), jnp.float32)], axis=0)
    bm = 640
    return pl.pallas_call(
        _outer_body,
        grid=(E // bm,),
        in_specs=[
            pl.BlockSpec((bm, 1), lambda i: (i, 0)),
            pl.BlockSpec((8, h), lambda i: (0, 0)),
        ],
        out_specs=pl.BlockSpec((bm, h), lambda i: (i, 0)),
        out_shape=jax.ShapeDtypeStruct((E, h), jnp.float32),
    )(e_w, wb8)


def _preph_body(ah_ref, num_ref, den_ref, h_ref, sn_ref, hn_ref, st_ref):
    i = pl.program_id(0)
    den = den_ref[...]
    safe = jnp.where(den == 0.0, 1.0, den)
    hagg = ah_ref[...] + num_ref[...] / safe
    mask = den[:, :1] > 0.0
    hnew = jnp.where(mask, hagg, h_ref[...])
    hp = hnew * sn_ref[...]
    hn_ref[...] = hp
    s = jnp.sum(hp, axis=0)
    s2 = jnp.sum(hp * hp, axis=0)
    upd = jnp.concatenate(
        [s[None, :], s2[None, :], jnp.zeros((6, s.shape[0]), jnp.float32)], axis=0)

    @pl.when(i == 0)
    def _():
        st_ref[...] = jnp.zeros_like(st_ref)

    st_ref[...] += upd


def _prep_h(ah, num, den, h, snorm_n):
    m, d = ah.shape
    bm = 400
    return pl.pallas_call(
        _preph_body,
        grid=(m // bm,),
        in_specs=[
            pl.BlockSpec((bm, d), lambda i: (i, 0)),
            pl.BlockSpec((bm, d), lambda i: (i, 0)),
            pl.BlockSpec((bm, d), lambda i: (i, 0)),
            pl.BlockSpec((bm, d), lambda i: (i, 0)),
            pl.BlockSpec((bm, 1), lambda i: (i, 0)),
        ],
        out_specs=[
            pl.BlockSpec((bm, d), lambda i: (i, 0)),
            pl.BlockSpec((8, d), lambda i: (0, 0)),
        ],
        out_shape=[
            jax.ShapeDtypeStruct((m, d), jnp.float32),
            jax.ShapeDtypeStruct((8, d), jnp.float32),
        ],
    )(ah, num, den, h, snorm_n)


def _statse_body(x_ref, rs_ref, st_ref):
    i = pl.program_id(0)
    y = x_ref[...] * rs_ref[...]
    s = jnp.sum(y, axis=0)
    s2 = jnp.sum(y * y, axis=0)
    upd = jnp.concatenate(
        [s[None, :], s2[None, :], jnp.zeros((6, s.shape[0]), jnp.float32)], axis=0)

    @pl.when(i == 0)
    def _():
        st_ref[...] = jnp.zeros_like(st_ref)

    st_ref[...] += upd


def _stats_rows(x, rowscale):
    m, d = x.shape
    bm = 640 if m == E else 400
    return pl.pallas_call(
        _statse_body,
        grid=(m // bm,),
        in_specs=[
            pl.BlockSpec((bm, d), lambda i: (i, 0)),
            pl.BlockSpec((bm, 1), lambda i: (i, 0)),
        ],
        out_specs=pl.BlockSpec((8, d), lambda i: (0, 0)),
        out_shape=jax.ShapeDtypeStruct((8, d), jnp.float32),
    )(x, rowscale)


def _apply_body(x_ref, rs_ref, res_ref, gb_ref, o_ref):
    y = x_ref[...] * rs_ref[...]
    y = y * gb_ref[0, :][None, :] + gb_ref[1, :][None, :]
    o_ref[...] = res_ref[...] + jnp.maximum(y, 0.0)


def _apply(x, rowscale, res, gs, bs):
    m, d = x.shape
    bm = 640 if m == E else 400
    gb = jnp.concatenate(
        [gs[None, :], bs[None, :], jnp.zeros((6, d), jnp.float32)], axis=0)
    return pl.pallas_call(
        _apply_body,
        grid=(m // bm,),
        in_specs=[
            pl.BlockSpec((bm, d), lambda i: (i, 0)),
            pl.BlockSpec((bm, 1), lambda i: (i, 0)),
            pl.BlockSpec((bm, d), lambda i: (i, 0)),
            pl.BlockSpec((8, d), lambda i: (0, 0)),
        ],
        out_specs=pl.BlockSpec((bm, d), lambda i: (i, 0)),
        out_shape=jax.ShapeDtypeStruct((m, d), jnp.float32),
    )(x, rowscale, res, gb)


_CW = 128


@functools.lru_cache(maxsize=None)
def _edge_kernel(d):
    nsl = d // _CW
    ng = _CW // _L
    per_tile = E // _NS
    nchunk = per_tile // _K

    mesh = plsc.VectorSubcoreMesh(core_axis_name="c", subcore_axis_name="s")

    @functools.partial(
        pl.kernel,
        out_type=[
            jax.ShapeDtypeStruct((E, nsl, _CW), jnp.float32),
            jax.ShapeDtypeStruct((N, nsl, _CW), jnp.float32),
            jax.ShapeDtypeStruct((N, nsl, _CW), jnp.float32),
        ],
        mesh=mesh,
        scratch_types=[
            pltpu.VMEM_SHARED((_NPAD, 1, _CW), jnp.float32),
            pltpu.VMEM((_K,), jnp.int32),
            pltpu.VMEM((_K,), jnp.int32),
            pltpu.VMEM((_K, 1, _CW), jnp.float32),
            pltpu.VMEM((_K, 1, _CW), jnp.float32),
            pltpu.VMEM((_K, 1, _CW), jnp.float32),
            pltpu.VMEM((_K, 1, _CW), jnp.float32),
            pltpu.SemaphoreType.DMA,
        ],
    )
    def kern(bh, dh, eh, ce, src, dst, eij_o, num_o, den_o,
             acc, sidx, didx, bd, be, bc, pay, sem):
        cid = jax.lax.axis_index("c")
        sid = jax.lax.axis_index("s")

        for p in range(nsl):
            j = cid * nsl + p
            is_num = (j % 2) == 0
            gq = j // 2
            if p > 0:
                plsc.subcore_barrier()

            def zb_row(r, _):
                for jg in range(ng):
                    pay[r, 0, pl.ds(jg * _L, _L)] = jnp.zeros((_L,), jnp.float32)
                return 0
            jax.lax.fori_loop(0, _K, zb_row, 0)
            for jz in range(_STRIPE // _K):
                pltpu.sync_copy(
                    pay, acc.at[pl.ds(sid * _STRIPE + jz * _K, _K)])
            plsc.subcore_barrier()

            def chunk(i, _):
                base = sid * per_tile + i * _K
                pltpu.sync_copy(src.at[pl.ds(base, _K)], sidx)
                pltpu.sync_copy(dst.at[pl.ds(base, _K)], didx)
                cp_d = pltpu.make_async_copy(dh.at[sidx, pl.ds(gq, 1)], bd, sem)
                cp_e = pltpu.make_async_copy(eh.at[didx, pl.ds(gq, 1)], be, sem)
                cp_c = pltpu.make_async_copy(ce.at[pl.ds(base, _K), pl.ds(gq, 1)],
                                             bc, sem)
                cp_d.start(); cp_e.start(); cp_c.start()

                @pl.when(is_num)
                def _():
                    pltpu.make_async_copy(bh.at[sidx, pl.ds(gq, 1)], pay, sem
                                          ).start()

                cp_d.wait(); cp_e.wait(); cp_c.wait()

                def row1(r, _):
                    for jg in range(ng):
                        sl = pl.ds(jg * _L, _L)
                        bc[r, 0, sl] = bc[r, 0, sl] + bd[r, 0, sl] + be[r, 0, sl]
                    return 0
                jax.lax.fori_loop(0, _K, row1, 0)

                @pl.when(is_num)
                def _():
                    pltpu.make_async_copy(bh.at[sidx, pl.ds(gq, 1)], pay, sem
                                          ).wait()
                    pltpu.sync_copy(bc, eij_o.at[pl.ds(base, _K), pl.ds(gq, 1)])

                def row2(r, _):
                    for jg in range(ng):
                        sl = pl.ds(jg * _L, _L)
                        sg = 1.0 / (1.0 + jnp.exp(-bc[r, 0, sl]))
                        sg = jnp.minimum(jnp.maximum(sg, 1e-4), 1.0 - 1e-4)
                        gate = jnp.where(is_num, pay[r, 0, sl],
                                         jnp.ones((_L,), jnp.float32))
                        pay[r, 0, sl] = sg * gate
                    return 0
                jax.lax.fori_loop(0, _K, row2, 0)

                pltpu.sync_copy(pay, acc.at[didx], add=True)
                return 0
            jax.lax.fori_loop(0, nchunk, chunk, 0)

            plsc.subcore_barrier()
            for jj in range(_STRIPE // _K):
                r0 = sid * _STRIPE + jj * _K
                @pl.when(jnp.logical_or(sid < _NS - 1, jj < 5))
                def _():
                    @pl.when(is_num)
                    def _():
                        pltpu.sync_copy(acc.at[pl.ds(r0, _K)],
                                        num_o.at[pl.ds(r0, _K), pl.ds(gq, 1)])

                    @pl.when(jnp.logical_not(is_num))
                    def _():
                        pltpu.sync_copy(acc.at[pl.ds(r0, _K)],
                                        den_o.at[pl.ds(r0, _K), pl.ds(gq, 1)])

    return kern


def _edge_phase(bh, dh, eh, ce, src, dst):
    d = bh.shape[1]
    nsl = d // _CW
    kern = _edge_kernel(d)
    eij, num, den = kern(
        bh.reshape(N, nsl, _CW), dh.reshape(N, nsl, _CW),
        eh.reshape(N, nsl, _CW), ce.reshape(E, nsl, _CW), src, dst)
    return eij.reshape(E, d), num.reshape(N, d), den.reshape(N, d)


def _bn_coeffs(st, m, g, b, eps=1e-5):
    mean = st[0, :] / m
    var = st[1, :] / m - mean * mean
    gs = g / jnp.sqrt(var + eps)
    return gs, b - mean * gs


def _gated_layer(h, e, src, dst, p, snorm_n, snorm_e, ones_n):
    ah = _mm(h, p['A_w'].T, p['A_b'])
    bhh = _mm(h, p['B_w'].T, p['B_b'])
    dhh = _mm(h, p['D_w'].T, p['D_b'])
    ehh = _mm(h, p['E_w'].T, p['E_b'])
    ce = _mm(e, p['C_w'].T, p['C_b'])
    eij, num, den = _edge_phase(bhh, dhh, ehh, ce, src, dst)
    hn_pre, hst = _prep_h(ah, num, den, h, snorm_n)
    est = _stats_rows(eij, snorm_e)
    gs_h, bs_h = _bn_coeffs(hst, N, p['bn_h_g'], p['bn_h_b'])
    gs_e, bs_e = _bn_coeffs(est, E, p['bn_e_g'], p['bn_e_b'])
    h_out = _apply(hn_pre, ones_n, h, gs_h, bs_h)
    e_out = _apply(eij, snorm_e, e, gs_e, bs_e)
    return h_out, e_out


def kernel(feats, e_w, snorm_n, snorm_e, edge_index, maps, params):
    src = edge_index[0]
    dst = edge_index[1]
    ones_n = jnp.ones((N, 1), jnp.float32)

    h = _mm(feats, params['emb_h_w'].T, params['emb_h_b'])
    e = _outer(e_w, params['emb_e_w'][:, 0], params['emb_e_b'])

    h, e = _gated_layer(h, e, src, dst, params['inp1'], snorm_n, snorm_e, ones_n)
    h, e_inp = _gated_layer(h, e, src, dst, params['inp2'], snorm_n, snorm_e, ones_n)

    z = jax.random.normal(jax.random.key(42), (N, ZD), dtype=jnp.float32)
    h_dec = jnp.concatenate([h, z], axis=-1)
    e_dec = _mm(e_inp, params['emb_edec_w'].T, params['emb_edec_b'])

    h2, e2 = _gated_layer(h_dec, e_dec, src, dst, params['dec1'], snorm_n,
                          snorm_e, ones_n)
    h2, e2 = _gated_layer(h2, e2, src, dst, params['dec2'], snorm_n,
                          snorm_e, ones_n)

    x = _mm(h2, params['mlp0_w'].T, params['mlp0_b'], relu=True)
    x = _mm(x, params['mlp1_w'].T, params['mlp1_b'], relu=True)
    return _mm(x, params['mlp2_w'].T, params['mlp2_b'])


# speedup vs baseline: 1.1527x; 1.1527x over previous
import functools

import jax
import jax.numpy as jnp
from jax.experimental import pallas as pl
from jax.experimental.pallas import tpu as pltpu
from jax.experimental.pallas import tpu_sc as plsc

N = 10000
E = 160000
HID = 256
ZD = 128
DEC = HID + ZD

_NS = 16
_L = 16
_K = 40
_NPAD = 10240
_STRIPE = _NPAD // _NS


def _mm_body(x_ref, w_ref, b_ref, o_ref, *, relu):
    acc = jnp.dot(x_ref[...], w_ref[...], preferred_element_type=jnp.float32)
    acc = acc + b_ref[0, :][None, :]
    if relu:
        acc = jnp.maximum(acc, 0.0)
    o_ref[...] = acc


def _mm(x, w, b, relu=False):
    m, k = x.shape
    n = w.shape[1]
    bm = 640 if m == E else 400
    b8 = jnp.broadcast_to(b[None, :], (8, n))
    return pl.pallas_call(
        functools.partial(_mm_body, relu=relu),
        grid=(m // bm,),
        in_specs=[
            pl.BlockSpec((bm, k), lambda i: (i, 0)),
            pl.BlockSpec((k, n), lambda i: (0, 0)),
            pl.BlockSpec((8, n), lambda i: (0, 0)),
        ],
        out_specs=pl.BlockSpec((bm, n), lambda i: (i, 0)),
        out_shape=jax.ShapeDtypeStruct((m, n), jnp.float32),
    )(x, w, b8)


def _outer_body(ew_ref, wb_ref, o_ref):
    o_ref[...] = ew_ref[...] * wb_ref[0, :][None, :] + wb_ref[1, :][None, :]


def _outer(e_w, w_col, b):
    h = w_col.shape[0]
    wb = jnp.concatenate([w_col[None, :], b[None, :]], axis=0)
    wb8 = jnp.concatenate([wb, jnp.zeros((6, h), jnp.float32)], axis=0)
    bm = 640
    return pl.pallas_call(
        _outer_body,
        grid=(E // bm,),
        in_specs=[
            pl.BlockSpec((bm, 1), lambda i: (i, 0)),
            pl.BlockSpec((8, h), lambda i: (0, 0)),
        ],
        out_specs=pl.BlockSpec((bm, h), lambda i: (i, 0)),
        out_shape=jax.ShapeDtypeStruct((E, h), jnp.float32),
    )(e_w, wb8)


def _preph_body(ah_ref, num_ref, den_ref, h_ref, sn_ref, hn_ref, st_ref):
    i = pl.program_id(0)
    den = den_ref[...]
    safe = jnp.where(den == 0.0, 1.0, den)
    hagg = ah_ref[...] + num_ref[...] / safe
    mask = den[:, :1] > 0.0
    hnew = jnp.where(mask, hagg, h_ref[...])
    hp = hnew * sn_ref[...]
    hn_ref[...] = hp
    s = jnp.sum(hp, axis=0)
    s2 = jnp.sum(hp * hp, axis=0)
    upd = jnp.concatenate(
        [s[None, :], s2[None, :], jnp.zeros((6, s.shape[0]), jnp.float32)], axis=0)

    @pl.when(i == 0)
    def _():
        st_ref[...] = jnp.zeros_like(st_ref)

    st_ref[...] += upd


def _prep_h(ah, num, den, h, snorm_n):
    m, d = ah.shape
    bm = 400
    return pl.pallas_call(
        _preph_body,
        grid=(m // bm,),
        in_specs=[
            pl.BlockSpec((bm, d), lambda i: (i, 0)),
            pl.BlockSpec((bm, d), lambda i: (i, 0)),
            pl.BlockSpec((bm, d), lambda i: (i, 0)),
            pl.BlockSpec((bm, d), lambda i: (i, 0)),
            pl.BlockSpec((bm, 1), lambda i: (i, 0)),
        ],
        out_specs=[
            pl.BlockSpec((bm, d), lambda i: (i, 0)),
            pl.BlockSpec((8, d), lambda i: (0, 0)),
        ],
        out_shape=[
            jax.ShapeDtypeStruct((m, d), jnp.float32),
            jax.ShapeDtypeStruct((8, d), jnp.float32),
        ],
    )(ah, num, den, h, snorm_n)


def _statse_body(x_ref, rs_ref, st_ref):
    i = pl.program_id(0)
    y = x_ref[...] * rs_ref[...]
    s = jnp.sum(y, axis=0)
    s2 = jnp.sum(y * y, axis=0)
    upd = jnp.concatenate(
        [s[None, :], s2[None, :], jnp.zeros((6, s.shape[0]), jnp.float32)], axis=0)

    @pl.when(i == 0)
    def _():
        st_ref[...] = jnp.zeros_like(st_ref)

    st_ref[...] += upd


def _stats_rows(x, rowscale):
    m, d = x.shape
    bm = 640 if m == E else 400
    return pl.pallas_call(
        _statse_body,
        grid=(m // bm,),
        in_specs=[
            pl.BlockSpec((bm, d), lambda i: (i, 0)),
            pl.BlockSpec((bm, 1), lambda i: (i, 0)),
        ],
        out_specs=pl.BlockSpec((8, d), lambda i: (0, 0)),
        out_shape=jax.ShapeDtypeStruct((8, d), jnp.float32),
    )(x, rowscale)


def _apply_body(x_ref, rs_ref, res_ref, gb_ref, o_ref):
    y = x_ref[...] * rs_ref[...]
    y = y * gb_ref[0, :][None, :] + gb_ref[1, :][None, :]
    o_ref[...] = res_ref[...] + jnp.maximum(y, 0.0)


def _apply(x, rowscale, res, gs, bs):
    m, d = x.shape
    bm = 640 if m == E else 400
    gb = jnp.concatenate(
        [gs[None, :], bs[None, :], jnp.zeros((6, d), jnp.float32)], axis=0)
    return pl.pallas_call(
        _apply_body,
        grid=(m // bm,),
        in_specs=[
            pl.BlockSpec((bm, d), lambda i: (i, 0)),
            pl.BlockSpec((bm, 1), lambda i: (i, 0)),
            pl.BlockSpec((bm, d), lambda i: (i, 0)),
            pl.BlockSpec((8, d), lambda i: (0, 0)),
        ],
        out_specs=pl.BlockSpec((bm, d), lambda i: (i, 0)),
        out_shape=jax.ShapeDtypeStruct((m, d), jnp.float32),
    )(x, rowscale, res, gb)


_CW = 128


@functools.lru_cache(maxsize=None)
def _edge_kernel(d):
    nsl = d // _CW
    ng = _CW // _L
    per_tile = E // _NS
    nchunk = per_tile // _K

    mesh = plsc.VectorSubcoreMesh(core_axis_name="c", subcore_axis_name="s")

    @functools.partial(
        pl.kernel,
        out_type=[
            jax.ShapeDtypeStruct((E, nsl, _CW), jnp.float32),
            jax.ShapeDtypeStruct((N, nsl, _CW), jnp.float32),
            jax.ShapeDtypeStruct((N, nsl, _CW), jnp.float32),
        ],
        mesh=mesh,
        scratch_types=[
            pltpu.VMEM_SHARED((_NPAD, 1, _CW), jnp.float32),
            pltpu.VMEM((_K,), jnp.int32),
            pltpu.VMEM((_K,), jnp.int32),
            pltpu.VMEM((_K,), jnp.int32),
            pltpu.VMEM((_K,), jnp.int32),
            pltpu.VMEM((_K, 1, _CW), jnp.float32),
            pltpu.VMEM((_K, 1, _CW), jnp.float32),
            pltpu.VMEM((_K, 1, _CW), jnp.float32),
            pltpu.VMEM((_K, 1, _CW), jnp.float32),
            pltpu.VMEM((_K, 1, _CW), jnp.float32),
            pltpu.VMEM((_K, 1, _CW), jnp.float32),
            pltpu.VMEM((_K, 1, _CW), jnp.float32),
            pltpu.VMEM((_K, 1, _CW), jnp.float32),
            pltpu.SemaphoreType.DMA,
            pltpu.SemaphoreType.DMA,
        ],
    )
    def kern(bh, dh, eh, ce, src, dst, eij_o, num_o, den_o,
             acc, sx0, dx0, sx1, dx1, bd0, be0, bc0, py0, bd1, be1, bc1, py1,
             sm0, sm1):
        cid = jax.lax.axis_index("c")
        sid = jax.lax.axis_index("s")
        sets = ((sx0, dx0, bd0, be0, bc0, py0, sm0),
                (sx1, dx1, bd1, be1, bc1, py1, sm1))

        for p in range(nsl):
            j = cid * nsl + p
            is_num = (j % 2) == 0
            gq = j // 2
            if p > 0:
                plsc.subcore_barrier()

            def zb_row(r, _):
                for jg in range(ng):
                    py0[r, 0, pl.ds(jg * _L, _L)] = jnp.zeros((_L,), jnp.float32)
                return 0
            jax.lax.fori_loop(0, _K, zb_row, 0)
            for jz in range(_STRIPE // _K):
                pltpu.sync_copy(
                    py0, acc.at[pl.ds(sid * _STRIPE + jz * _K, _K)])
            plsc.subcore_barrier()

            def gath(tbl, idxrow, buf, sem):
                return pltpu.make_async_copy(tbl.at[idxrow, pl.ds(gq, 1)],
                                             buf, sem)

            def issue(i, s):
                sx, dx, bd, be, bc, py, sem = sets[s]
                base = sid * per_tile + i * _K
                pltpu.sync_copy(src.at[pl.ds(base, _K)], sx)
                pltpu.sync_copy(dst.at[pl.ds(base, _K)], dx)
                gath(dh, sx, bd, sem).start()
                gath(eh, dx, be, sem).start()
                pltpu.make_async_copy(ce.at[pl.ds(base, _K), pl.ds(gq, 1)],
                                      bc, sem).start()

                @pl.when(is_num)
                def _():
                    gath(bh, sx, py, sem).start()

            def finish(i, s):
                sx, dx, bd, be, bc, py, sem = sets[s]
                base = sid * per_tile + i * _K
                gath(dh, sx, bd, sem).wait()
                gath(eh, dx, be, sem).wait()
                pltpu.make_async_copy(ce.at[pl.ds(base, _K), pl.ds(gq, 1)],
                                      bc, sem).wait()

                @pl.when(is_num)
                def _():
                    gath(bh, sx, py, sem).wait()

                def rows(r2, _):
                    for u in range(2):
                        r = r2 * 2 + u
                        for jg in range(ng):
                            sl = pl.ds(jg * _L, _L)
                            eij = bc[r, 0, sl] + bd[r, 0, sl] + be[r, 0, sl]
                            sg = 1.0 / (1.0 + jnp.exp(-eij))
                            sg = jnp.minimum(jnp.maximum(sg, 1e-4), 1.0 - 1e-4)
                            bc[r, 0, sl] = eij
                            gate = jnp.where(is_num, py[r, 0, sl],
                                             jnp.ones((_L,), jnp.float32))
                            py[r, 0, sl] = sg * gate
                    return 0
                jax.lax.fori_loop(0, _K // 2, rows, 0)

                @pl.when(is_num)
                def _():
                    pltpu.sync_copy(bc, eij_o.at[pl.ds(base, _K), pl.ds(gq, 1)])

                pltpu.sync_copy(py, acc.at[dx], add=True)

            issue(0, 0)

            def pair(ii, _):
                i0 = ii * 2
                issue(i0 + 1, 1)
                finish(i0, 0)

                @pl.when(i0 + 2 < nchunk)
                def _():
                    issue(i0 + 2, 0)

                finish(i0 + 1, 1)
                return 0
            jax.lax.fori_loop(0, nchunk // 2, pair, 0)

            plsc.subcore_barrier()
            last_valid = (N - (_NS - 1) * _STRIPE) // _K
            for jj in range(_STRIPE // _K):
                r0 = sid * _STRIPE + jj * _K
                @pl.when(jnp.logical_or(sid < _NS - 1, jj < last_valid))
                def _():
                    @pl.when(is_num)
                    def _():
                        pltpu.sync_copy(acc.at[pl.ds(r0, _K)],
                                        num_o.at[pl.ds(r0, _K), pl.ds(gq, 1)])

                    @pl.when(jnp.logical_not(is_num))
                    def _():
                        pltpu.sync_copy(acc.at[pl.ds(r0, _K)],
                                        den_o.at[pl.ds(r0, _K), pl.ds(gq, 1)])

    return kern


def _edge_phase(bh, dh, eh, ce, ei):
    d = bh.shape[1]
    nsl = d // _CW
    kern = _edge_kernel(d)
    eij, num, den = kern(
        bh.reshape(N, nsl, _CW), dh.reshape(N, nsl, _CW),
        eh.reshape(N, nsl, _CW), ce.reshape(E, nsl, _CW), ei[0], ei[1])
    return eij.reshape(E, d), num.reshape(N, d), den.reshape(N, d)


def _bn_coeffs(st, m, g, b, eps=1e-5):
    mean = st[0, :] / m
    var = st[1, :] / m - mean * mean
    gs = g / jnp.sqrt(var + eps)
    return gs, b - mean * gs


def _gated_layer(h, e, ei, p, snorm_n, snorm_e, ones_n):
    ah = _mm(h, p['A_w'].T, p['A_b'])
    bhh = _mm(h, p['B_w'].T, p['B_b'])
    dhh = _mm(h, p['D_w'].T, p['D_b'])
    ehh = _mm(h, p['E_w'].T, p['E_b'])
    ce = _mm(e, p['C_w'].T, p['C_b'])
    eij, num, den = _edge_phase(bhh, dhh, ehh, ce, ei)
    hn_pre, hst = _prep_h(ah, num, den, h, snorm_n)
    est = _stats_rows(eij, snorm_e)
    gs_h, bs_h = _bn_coeffs(hst, N, p['bn_h_g'], p['bn_h_b'])
    gs_e, bs_e = _bn_coeffs(est, E, p['bn_e_g'], p['bn_e_b'])
    h_out = _apply(hn_pre, ones_n, h, gs_h, bs_h)
    e_out = _apply(eij, snorm_e, e, gs_e, bs_e)
    return h_out, e_out


def kernel(feats, e_w, snorm_n, snorm_e, edge_index, maps, params):
    ei = edge_index
    ones_n = jnp.ones((N, 1), jnp.float32)

    h = _mm(feats, params['emb_h_w'].T, params['emb_h_b'])
    e = _outer(e_w, params['emb_e_w'][:, 0], params['emb_e_b'])

    h, e = _gated_layer(h, e, ei, params['inp1'], snorm_n, snorm_e, ones_n)
    h, e_inp = _gated_layer(h, e, ei, params['inp2'], snorm_n, snorm_e, ones_n)

    z = jax.random.normal(jax.random.key(42), (N, ZD), dtype=jnp.float32)
    h_dec = jnp.concatenate([h, z], axis=-1)
    e_dec = _mm(e_inp, params['emb_edec_w'].T, params['emb_edec_b'])

    h2, e2 = _gated_layer(h_dec, e_dec, ei, params['dec1'], snorm_n,
                          snorm_e, ones_n)
    h2, e2 = _gated_layer(h2, e2, ei, params['dec2'], snorm_n,
                          snorm_e, ones_n)

    x = _mm(h2, params['mlp0_w'].T, params['mlp0_b'], relu=True)
    x = _mm(x, params['mlp1_w'].T, params['mlp1_b'], relu=True)
    return _mm(x, params['mlp2_w'].T, params['mlp2_b'])


# speedup vs baseline: 1.3586x; 1.1786x over previous
import functools

import jax
import jax.numpy as jnp
from jax.experimental import pallas as pl
from jax.experimental.pallas import tpu as pltpu
from jax.experimental.pallas import tpu_sc as plsc

N = 10000
E = 160000
HID = 256
ZD = 128
DEC = HID + ZD

_NS = 16
_L = 16
_K = 40
_NPAD = 10240
_STRIPE = _NPAD // _NS


def _mm(x, w, b, relu=False):
    def body(x_ref, w_ref, b_ref, o_ref):
        acc = jnp.dot(x_ref[...], w_ref[...], preferred_element_type=jnp.float32)
        acc = acc + b_ref[0, :][None, :]
        if relu:
            acc = jnp.maximum(acc, 0.0)
        o_ref[...] = acc

    m, k = x.shape
    n = w.shape[1]
    bm = 640 if m == E else 400
    b8 = jnp.broadcast_to(b[None, :], (8, n))
    return pl.pallas_call(
        body,
        grid=(m // bm,),
        in_specs=[
            pl.BlockSpec((bm, k), lambda i: (i, 0)),
            pl.BlockSpec((k, n), lambda i: (0, 0)),
            pl.BlockSpec((8, n), lambda i: (0, 0)),
        ],
        out_specs=pl.BlockSpec((bm, n), lambda i: (i, 0)),
        out_shape=jax.ShapeDtypeStruct((m, n), jnp.float32),
    )(x, w, b8)


def _mm3(x3, w3, b, relu=False, out3=True):
    m, ki, _ = x3.shape
    n = w3.shape[2]
    no = n // _CW
    bm = 640 if m == E else 400
    b8 = jnp.broadcast_to(b[None, :], (8, n))

    def body(x_ref, w_ref, b_ref, o_ref):
        acc = jnp.dot(x_ref[:, 0, :], w_ref[0],
                      preferred_element_type=jnp.float32)
        for s in range(1, ki):
            acc += jnp.dot(x_ref[:, s, :], w_ref[s],
                           preferred_element_type=jnp.float32)
        acc = acc + b_ref[0, :][None, :]
        if relu:
            acc = jnp.maximum(acc, 0.0)
        if out3:
            for t in range(no):
                o_ref[:, t, :] = acc[:, t * _CW:(t + 1) * _CW]
        else:
            o_ref[...] = acc

    if out3:
        out_spec = pl.BlockSpec((bm, no, _CW), lambda i: (i, 0, 0))
        out_shape = jax.ShapeDtypeStruct((m, no, _CW), jnp.float32)
    else:
        out_spec = pl.BlockSpec((bm, n), lambda i: (i, 0))
        out_shape = jax.ShapeDtypeStruct((m, n), jnp.float32)
    return pl.pallas_call(
        body,
        grid=(m // bm,),
        in_specs=[
            pl.BlockSpec((bm, ki, _CW), lambda i: (i, 0, 0)),
            pl.BlockSpec((ki, _CW, n), lambda i: (0, 0, 0)),
            pl.BlockSpec((8, n), lambda i: (0, 0)),
        ],
        out_specs=out_spec,
        out_shape=out_shape,
    )(x3, w3, b8)


def _mm_node(h3, w3, b, d):
    m, ki, _ = h3.shape
    no = d // _CW
    bm = 400
    b8 = jnp.broadcast_to(b[None, :], (8, 4 * d))

    def body(x_ref, w_ref, b_ref, oa, ob, od, oe):
        acc = jnp.dot(x_ref[:, 0, :], w_ref[0],
                      preferred_element_type=jnp.float32)
        for s in range(1, ki):
            acc += jnp.dot(x_ref[:, s, :], w_ref[s],
                           preferred_element_type=jnp.float32)
        acc = acc + b_ref[0, :][None, :]
        for q, o_ref in enumerate((oa, ob, od, oe)):
            for t in range(no):
                c0 = q * d + t * _CW
                o_ref[:, t, :] = acc[:, c0:c0 + _CW]

    spec = pl.BlockSpec((bm, no, _CW), lambda i: (i, 0, 0))
    shp = jax.ShapeDtypeStruct((m, no, _CW), jnp.float32)
    return pl.pallas_call(
        body,
        grid=(m // bm,),
        in_specs=[
            pl.BlockSpec((bm, ki, _CW), lambda i: (i, 0, 0)),
            pl.BlockSpec((ki, _CW, 4 * d), lambda i: (0, 0, 0)),
            pl.BlockSpec((8, 4 * d), lambda i: (0, 0)),
        ],
        out_specs=[spec, spec, spec, spec],
        out_shape=[shp, shp, shp, shp],
    )(h3, w3, b8)


def _outer(e_w, w_col, b):
    hdim = w_col.shape[0]
    no = hdim // _CW
    wb = jnp.concatenate([w_col[None, :], b[None, :]], axis=0)
    wb8 = jnp.concatenate([wb, jnp.zeros((6, hdim), jnp.float32)], axis=0)
    bm = 640

    def body(ew_ref, wb_ref, o_ref):
        v = ew_ref[...] * wb_ref[0, :][None, :] + wb_ref[1, :][None, :]
        for t in range(no):
            o_ref[:, t, :] = v[:, t * _CW:(t + 1) * _CW]

    return pl.pallas_call(
        body,
        grid=(E // bm,),
        in_specs=[
            pl.BlockSpec((bm, 1), lambda i: (i, 0)),
            pl.BlockSpec((8, hdim), lambda i: (0, 0)),
        ],
        out_specs=pl.BlockSpec((bm, no, _CW), lambda i: (i, 0, 0)),
        out_shape=jax.ShapeDtypeStruct((E, no, _CW), jnp.float32),
    )(e_w, wb8)


def _preph_body(ah_ref, num_ref, den_ref, h_ref, sn_ref, hn_ref, st_ref):
    i = pl.program_id(0)
    den = den_ref[...]
    safe = jnp.where(den == 0.0, 1.0, den)
    hagg = ah_ref[...] + num_ref[...] / safe
    mask = den[:, :1, :1] > 0.0
    hnew = jnp.where(mask, hagg, h_ref[...])
    hp = hnew * sn_ref[...][:, :, None]
    hn_ref[...] = hp
    s = jnp.sum(hp, axis=0)
    s2 = jnp.sum(hp * hp, axis=0)
    upd = jnp.concatenate(
        [s[None], s2[None],
         jnp.zeros((6,) + s.shape, jnp.float32)], axis=0)

    @pl.when(i == 0)
    def _():
        st_ref[...] = jnp.zeros_like(st_ref)

    st_ref[...] += upd


def _prep_h(ah, num, den, h, snorm_n):
    m, nsl, _ = ah.shape
    bm = 400
    spec = pl.BlockSpec((bm, nsl, _CW), lambda i: (i, 0, 0))
    return pl.pallas_call(
        _preph_body,
        grid=(m // bm,),
        in_specs=[spec, spec, spec, spec,
                  pl.BlockSpec((bm, 1), lambda i: (i, 0))],
        out_specs=[spec, pl.BlockSpec((8, nsl, _CW), lambda i: (0, 0, 0))],
        out_shape=[
            jax.ShapeDtypeStruct((m, nsl, _CW), jnp.float32),
            jax.ShapeDtypeStruct((8, nsl, _CW), jnp.float32),
        ],
    )(ah, num, den, h, snorm_n)


def _statse_body(x_ref, rs_ref, st_ref):
    i = pl.program_id(0)
    y = x_ref[...] * rs_ref[...][:, :, None]
    s = jnp.sum(y, axis=0)
    s2 = jnp.sum(y * y, axis=0)
    upd = jnp.concatenate(
        [s[None], s2[None],
         jnp.zeros((6,) + s.shape, jnp.float32)], axis=0)

    @pl.when(i == 0)
    def _():
        st_ref[...] = jnp.zeros_like(st_ref)

    st_ref[...] += upd


def _stats_rows(x, rowscale):
    m, nsl, _ = x.shape
    bm = 640 if m == E else 400
    spec = pl.BlockSpec((bm, nsl, _CW), lambda i: (i, 0, 0))
    return pl.pallas_call(
        _statse_body,
        grid=(m // bm,),
        in_specs=[spec, pl.BlockSpec((bm, 1), lambda i: (i, 0))],
        out_specs=pl.BlockSpec((8, nsl, _CW), lambda i: (0, 0, 0)),
        out_shape=jax.ShapeDtypeStruct((8, nsl, _CW), jnp.float32),
    )(x, rowscale)


def _apply_body(x_ref, rs_ref, res_ref, gb_ref, o_ref):
    y = x_ref[...] * rs_ref[...][:, :, None]
    y = y * gb_ref[0][None] + gb_ref[1][None]
    o_ref[...] = res_ref[...] + jnp.maximum(y, 0.0)


def _apply(x, rowscale, res, gb):
    m, nsl, _ = x.shape
    bm = 640 if m == E else 400
    spec = pl.BlockSpec((bm, nsl, _CW), lambda i: (i, 0, 0))
    return pl.pallas_call(
        _apply_body,
        grid=(m // bm,),
        in_specs=[spec, pl.BlockSpec((bm, 1), lambda i: (i, 0)),
                  spec, pl.BlockSpec((8, nsl, _CW), lambda i: (0, 0, 0))],
        out_specs=spec,
        out_shape=jax.ShapeDtypeStruct((m, nsl, _CW), jnp.float32),
    )(x, rowscale, res, gb)


_CW = 128


@functools.lru_cache(maxsize=None)
def _edge_kernel(d):
    nsl = d // _CW
    ng = _CW // _L
    per_tile = E // _NS
    nchunk = per_tile // _K

    mesh = plsc.VectorSubcoreMesh(core_axis_name="c", subcore_axis_name="s")

    @functools.partial(
        pl.kernel,
        out_type=[
            jax.ShapeDtypeStruct((E, nsl, _CW), jnp.float32),
            jax.ShapeDtypeStruct((N, nsl, _CW), jnp.float32),
            jax.ShapeDtypeStruct((N, nsl, _CW), jnp.float32),
        ],
        mesh=mesh,
        scratch_types=[
            pltpu.VMEM_SHARED((_NPAD, 1, _CW), jnp.float32),
            pltpu.VMEM((_K,), jnp.int32),
            pltpu.VMEM((_K,), jnp.int32),
            pltpu.VMEM((_K,), jnp.int32),
            pltpu.VMEM((_K,), jnp.int32),
            pltpu.VMEM((_K, 1, _CW), jnp.float32),
            pltpu.VMEM((_K, 1, _CW), jnp.float32),
            pltpu.VMEM((_K, 1, _CW), jnp.float32),
            pltpu.VMEM((_K, 1, _CW), jnp.float32),
            pltpu.VMEM((_K, 1, _CW), jnp.float32),
            pltpu.VMEM((_K, 1, _CW), jnp.float32),
            pltpu.VMEM((_K, 1, _CW), jnp.float32),
            pltpu.VMEM((_K, 1, _CW), jnp.float32),
            pltpu.SemaphoreType.DMA,
            pltpu.SemaphoreType.DMA,
        ],
    )
    def kern(bh, dh, eh, ce, src, dst, eij_o, num_o, den_o,
             acc, sx0, dx0, sx1, dx1, bd0, be0, bc0, py0, bd1, be1, bc1, py1,
             sm0, sm1):
        cid = jax.lax.axis_index("c")
        sid = jax.lax.axis_index("s")
        sets = ((sx0, dx0, bd0, be0, bc0, py0, sm0),
                (sx1, dx1, bd1, be1, bc1, py1, sm1))

        for p in range(nsl):
            j = cid * nsl + p
            is_num = (j % 2) == 0
            gq = j // 2
            if p > 0:
                plsc.subcore_barrier()

            def zb_row(r, _):
                for jg in range(ng):
                    py0[r, 0, pl.ds(jg * _L, _L)] = jnp.zeros((_L,), jnp.float32)
                return 0
            jax.lax.fori_loop(0, _K, zb_row, 0)
            for jz in range(_STRIPE // _K):
                pltpu.sync_copy(
                    py0, acc.at[pl.ds(sid * _STRIPE + jz * _K, _K)])
            plsc.subcore_barrier()

            def gath(tbl, idxrow, buf, sem):
                return pltpu.make_async_copy(tbl.at[idxrow, pl.ds(gq, 1)],
                                             buf, sem)

            def issue(i, s):
                sx, dx, bd, be, bc, py, sem = sets[s]
                base = sid * per_tile + i * _K
                pltpu.sync_copy(src.at[pl.ds(base, _K)], sx)
                pltpu.sync_copy(dst.at[pl.ds(base, _K)], dx)
                gath(dh, sx, bd, sem).start()
                gath(eh, dx, be, sem).start()
                pltpu.make_async_copy(ce.at[pl.ds(base, _K), pl.ds(gq, 1)],
                                      bc, sem).start()

                @pl.when(is_num)
                def _():
                    gath(bh, sx, py, sem).start()

            def finish(i, s):
                sx, dx, bd, be, bc, py, sem = sets[s]
                base = sid * per_tile + i * _K
                gath(dh, sx, bd, sem).wait()
                gath(eh, dx, be, sem).wait()
                pltpu.make_async_copy(ce.at[pl.ds(base, _K), pl.ds(gq, 1)],
                                      bc, sem).wait()

                @pl.when(is_num)
                def _():
                    gath(bh, sx, py, sem).wait()

                def rows(r2, _):
                    for u in range(2):
                        r = r2 * 2 + u
                        for jg in range(ng):
                            sl = pl.ds(jg * _L, _L)
                            eij = bc[r, 0, sl] + bd[r, 0, sl] + be[r, 0, sl]
                            sg = 1.0 / (1.0 + jnp.exp(-eij))
                            sg = jnp.minimum(jnp.maximum(sg, 1e-4), 1.0 - 1e-4)
                            bc[r, 0, sl] = eij
                            gate = jnp.where(is_num, py[r, 0, sl],
                                             jnp.ones((_L,), jnp.float32))
                            py[r, 0, sl] = sg * gate
                    return 0
                jax.lax.fori_loop(0, _K // 2, rows, 0)

                @pl.when(is_num)
                def _():
                    pltpu.sync_copy(bc, eij_o.at[pl.ds(base, _K), pl.ds(gq, 1)])

                pltpu.sync_copy(py, acc.at[dx], add=True)

            issue(0, 0)

            def pair(ii, _):
                i0 = ii * 2
                issue(i0 + 1, 1)
                finish(i0, 0)

                @pl.when(i0 + 2 < nchunk)
                def _():
                    issue(i0 + 2, 0)

                finish(i0 + 1, 1)
                return 0
            jax.lax.fori_loop(0, nchunk // 2, pair, 0)

            plsc.subcore_barrier()
            last_valid = (N - (_NS - 1) * _STRIPE) // _K
            for jj in range(_STRIPE // _K):
                r0 = sid * _STRIPE + jj * _K
                @pl.when(jnp.logical_or(sid < _NS - 1, jj < last_valid))
                def _():
                    @pl.when(is_num)
                    def _():
                        pltpu.sync_copy(acc.at[pl.ds(r0, _K)],
                                        num_o.at[pl.ds(r0, _K), pl.ds(gq, 1)])

                    @pl.when(jnp.logical_not(is_num))
                    def _():
                        pltpu.sync_copy(acc.at[pl.ds(r0, _K)],
                                        den_o.at[pl.ds(r0, _K), pl.ds(gq, 1)])

    return kern


def _edge_phase(bh3, dh3, eh3, ce3, ei):
    d = bh3.shape[1] * _CW
    kern = _edge_kernel(d)
    return kern(bh3, dh3, eh3, ce3, ei[0], ei[1])


def _bn_gb(st, m, g, b, eps=1e-5):
    mean = st[0] / m
    var = st[1] / m - mean * mean
    gs = g.reshape(mean.shape) / jnp.sqrt(var + eps)
    bs = b.reshape(mean.shape) - mean * gs
    return jnp.concatenate(
        [gs[None], bs[None], jnp.zeros((6,) + mean.shape, jnp.float32)], axis=0)


def _w3(w):
    k, n = w.shape
    return w.reshape(k // _CW, _CW, n)


def _gated_layer(h3, e3, ei, p, snorm_n, snorm_e, ones_n):
    d = h3.shape[1] * _CW
    wcat = _w3(jnp.concatenate(
        [p['A_w'].T, p['B_w'].T, p['D_w'].T, p['E_w'].T], axis=1))
    bcat = jnp.concatenate([p['A_b'], p['B_b'], p['D_b'], p['E_b']])
    ah3, bh3, dh3, eh3 = _mm_node(h3, wcat, bcat, d)
    ce3 = _mm3(e3, _w3(p['C_w'].T), p['C_b'])
    eij3, num3, den3 = _edge_phase(bh3, dh3, eh3, ce3, ei)
    hn3, hst = _prep_h(ah3, num3, den3, h3, snorm_n)
    est = _stats_rows(eij3, snorm_e)
    gb_h = _bn_gb(hst, N, p['bn_h_g'], p['bn_h_b'])
    gb_e = _bn_gb(est, E, p['bn_e_g'], p['bn_e_b'])
    h_out = _apply(hn3, ones_n, h3, gb_h)
    e_out = _apply(eij3, snorm_e, e3, gb_e)
    return h_out, e_out


def kernel(feats, e_w, snorm_n, snorm_e, edge_index, maps, params):
    ei = edge_index
    ones_n = jnp.ones((N, 1), jnp.float32)

    h = _mm3(feats.reshape(N, HID // _CW, _CW), _w3(params['emb_h_w'].T),
             params['emb_h_b'])
    e = _outer(e_w, params['emb_e_w'][:, 0], params['emb_e_b'])

    h, e = _gated_layer(h, e, ei, params['inp1'], snorm_n, snorm_e, ones_n)
    h, e_inp = _gated_layer(h, e, ei, params['inp2'], snorm_n, snorm_e, ones_n)

    z = jax.random.normal(jax.random.key(42), (N, ZD), dtype=jnp.float32)
    h_dec = jnp.concatenate([h, z[:, None, :]], axis=1)
    e_dec = _mm3(e_inp, _w3(params['emb_edec_w'].T), params['emb_edec_b'])

    h2, e2 = _gated_layer(h_dec, e_dec, ei, params['dec1'], snorm_n,
                          snorm_e, ones_n)
    h2, e2 = _gated_layer(h2, e2, ei, params['dec2'], snorm_n,
                          snorm_e, ones_n)

    x = _mm3(h2, _w3(params['mlp0_w'].T), params['mlp0_b'], relu=True,
             out3=False)
    x = _mm(x, params['mlp1_w'].T, params['mlp1_b'], relu=True)
    return _mm(x, params['mlp2_w'].T, params['mlp2_b'])


# speedup vs baseline: 1.3963x; 1.0277x over previous
import functools

import jax
import jax.numpy as jnp
from jax.experimental import pallas as pl
from jax.experimental.pallas import tpu as pltpu
from jax.experimental.pallas import tpu_sc as plsc

N = 10000
E = 160000
HID = 256
ZD = 128
DEC = HID + ZD

_NS = 16
_L = 16
_K = 40
_NPAD = 10240
_STRIPE = _NPAD // _NS


def _mm(x, w, b, relu=False):
    def body(x_ref, w_ref, b_ref, o_ref):
        acc = jnp.dot(x_ref[...], w_ref[...], preferred_element_type=jnp.float32)
        acc = acc + b_ref[0, :][None, :]
        if relu:
            acc = jnp.maximum(acc, 0.0)
        o_ref[...] = acc

    m, k = x.shape
    n = w.shape[1]
    bm = 640 if m == E else 400
    b8 = jnp.broadcast_to(b[None, :], (8, n))
    return pl.pallas_call(
        body,
        grid=(m // bm,),
        in_specs=[
            pl.BlockSpec((bm, k), lambda i: (i, 0)),
            pl.BlockSpec((k, n), lambda i: (0, 0)),
            pl.BlockSpec((8, n), lambda i: (0, 0)),
        ],
        out_specs=pl.BlockSpec((bm, n), lambda i: (i, 0)),
        out_shape=jax.ShapeDtypeStruct((m, n), jnp.float32),
    )(x, w, b8)


def _mm3(x3, w3, b, relu=False, out3=True):
    m, ki, _ = x3.shape
    n = w3.shape[2]
    no = n // _CW
    bm = 640 if m == E else 400
    b8 = jnp.broadcast_to(b[None, :], (8, n))

    def body(x_ref, w_ref, b_ref, o_ref):
        acc = jnp.dot(x_ref[:, 0, :], w_ref[0],
                      preferred_element_type=jnp.float32)
        for s in range(1, ki):
            acc += jnp.dot(x_ref[:, s, :], w_ref[s],
                           preferred_element_type=jnp.float32)
        acc = acc + b_ref[0, :][None, :]
        if relu:
            acc = jnp.maximum(acc, 0.0)
        if out3:
            for t in range(no):
                o_ref[:, t, :] = acc[:, t * _CW:(t + 1) * _CW]
        else:
            o_ref[...] = acc

    if out3:
        out_spec = pl.BlockSpec((bm, no, _CW), lambda i: (i, 0, 0))
        out_shape = jax.ShapeDtypeStruct((m, no, _CW), jnp.float32)
    else:
        out_spec = pl.BlockSpec((bm, n), lambda i: (i, 0))
        out_shape = jax.ShapeDtypeStruct((m, n), jnp.float32)
    return pl.pallas_call(
        body,
        grid=(m // bm,),
        in_specs=[
            pl.BlockSpec((bm, ki, _CW), lambda i: (i, 0, 0)),
            pl.BlockSpec((ki, _CW, n), lambda i: (0, 0, 0)),
            pl.BlockSpec((8, n), lambda i: (0, 0)),
        ],
        out_specs=out_spec,
        out_shape=out_shape,
    )(x3, w3, b8)


def _mm_node(h3, w3, b, d):
    m, ki, _ = h3.shape
    no = d // _CW
    bm = 400
    b8 = jnp.broadcast_to(b[None, :], (8, 4 * d))

    def body(x_ref, w_ref, b_ref, oa, ob, od, oe):
        acc = jnp.dot(x_ref[:, 0, :], w_ref[0],
                      preferred_element_type=jnp.float32)
        for s in range(1, ki):
            acc += jnp.dot(x_ref[:, s, :], w_ref[s],
                           preferred_element_type=jnp.float32)
        acc = acc + b_ref[0, :][None, :]
        for q, o_ref in enumerate((oa, ob, od, oe)):
            for t in range(no):
                c0 = q * d + t * _CW
                o_ref[:, t, :] = acc[:, c0:c0 + _CW]

    spec = pl.BlockSpec((bm, no, _CW), lambda i: (i, 0, 0))
    shp = jax.ShapeDtypeStruct((m, no, _CW), jnp.float32)
    return pl.pallas_call(
        body,
        grid=(m // bm,),
        in_specs=[
            pl.BlockSpec((bm, ki, _CW), lambda i: (i, 0, 0)),
            pl.BlockSpec((ki, _CW, 4 * d), lambda i: (0, 0, 0)),
            pl.BlockSpec((8, 4 * d), lambda i: (0, 0)),
        ],
        out_specs=[spec, spec, spec, spec],
        out_shape=[shp, shp, shp, shp],
    )(h3, w3, b8)


def _outer(e_w, w_col, b):
    hdim = w_col.shape[0]
    no = hdim // _CW
    wb = jnp.concatenate([w_col[None, :], b[None, :]], axis=0)
    wb8 = jnp.concatenate([wb, jnp.zeros((6, hdim), jnp.float32)], axis=0)
    bm = 640

    def body(ew_ref, wb_ref, o_ref):
        v = ew_ref[...] * wb_ref[0, :][None, :] + wb_ref[1, :][None, :]
        for t in range(no):
            o_ref[:, t, :] = v[:, t * _CW:(t + 1) * _CW]

    return pl.pallas_call(
        body,
        grid=(E // bm,),
        in_specs=[
            pl.BlockSpec((bm, 1), lambda i: (i, 0)),
            pl.BlockSpec((8, hdim), lambda i: (0, 0)),
        ],
        out_specs=pl.BlockSpec((bm, no, _CW), lambda i: (i, 0, 0)),
        out_shape=jax.ShapeDtypeStruct((E, no, _CW), jnp.float32),
    )(e_w, wb8)


def _preph_body(ah_ref, num_ref, den_ref, h_ref, sn_ref, hn_ref, st_ref):
    i = pl.program_id(0)
    den = den_ref[...]
    safe = jnp.where(den == 0.0, 1.0, den)
    hagg = ah_ref[...] + num_ref[...] / safe
    mask = den[:, :1, :1] > 0.0
    hnew = jnp.where(mask, hagg, h_ref[...])
    hp = hnew * sn_ref[...][:, :, None]
    hn_ref[...] = hp
    s = jnp.sum(hp, axis=0)
    s2 = jnp.sum(hp * hp, axis=0)
    upd = jnp.concatenate(
        [s[None], s2[None],
         jnp.zeros((6,) + s.shape, jnp.float32)], axis=0)

    @pl.when(i == 0)
    def _():
        st_ref[...] = jnp.zeros_like(st_ref)

    st_ref[...] += upd


def _prep_h(ah, num, den, h, snorm_n):
    m, nsl, _ = ah.shape
    bm = 400
    spec = pl.BlockSpec((bm, nsl, _CW), lambda i: (i, 0, 0))
    return pl.pallas_call(
        _preph_body,
        grid=(m // bm,),
        in_specs=[spec, spec, spec, spec,
                  pl.BlockSpec((bm, 1), lambda i: (i, 0))],
        out_specs=[spec, pl.BlockSpec((8, nsl, _CW), lambda i: (0, 0, 0))],
        out_shape=[
            jax.ShapeDtypeStruct((m, nsl, _CW), jnp.float32),
            jax.ShapeDtypeStruct((8, nsl, _CW), jnp.float32),
        ],
    )(ah, num, den, h, snorm_n)


def _statse_body(x_ref, rs_ref, st_ref):
    i = pl.program_id(0)
    y = x_ref[...] * rs_ref[...][:, :, None]
    s = jnp.sum(y, axis=0)
    s2 = jnp.sum(y * y, axis=0)
    upd = jnp.concatenate(
        [s[None], s2[None],
         jnp.zeros((6,) + s.shape, jnp.float32)], axis=0)

    @pl.when(i == 0)
    def _():
        st_ref[...] = jnp.zeros_like(st_ref)

    st_ref[...] += upd


def _stats_rows(x, rowscale):
    m, nsl, _ = x.shape
    bm = 640 if m == E else 400
    spec = pl.BlockSpec((bm, nsl, _CW), lambda i: (i, 0, 0))
    return pl.pallas_call(
        _statse_body,
        grid=(m // bm,),
        in_specs=[spec, pl.BlockSpec((bm, 1), lambda i: (i, 0))],
        out_specs=pl.BlockSpec((8, nsl, _CW), lambda i: (0, 0, 0)),
        out_shape=jax.ShapeDtypeStruct((8, nsl, _CW), jnp.float32),
    )(x, rowscale)


def _apply_body(x_ref, rs_ref, res_ref, gb_ref, o_ref):
    y = x_ref[...] * rs_ref[...][:, :, None]
    y = y * gb_ref[0][None] + gb_ref[1][None]
    o_ref[...] = res_ref[...] + jnp.maximum(y, 0.0)


def _apply(x, rowscale, res, gb):
    m, nsl, _ = x.shape
    bm = 640 if m == E else 400
    spec = pl.BlockSpec((bm, nsl, _CW), lambda i: (i, 0, 0))
    return pl.pallas_call(
        _apply_body,
        grid=(m // bm,),
        in_specs=[spec, pl.BlockSpec((bm, 1), lambda i: (i, 0)),
                  spec, pl.BlockSpec((8, nsl, _CW), lambda i: (0, 0, 0))],
        out_specs=spec,
        out_shape=jax.ShapeDtypeStruct((m, nsl, _CW), jnp.float32),
    )(x, rowscale, res, gb)


_CW = 128


@functools.lru_cache(maxsize=None)
def _edge_kernel(d, write_eij=True):
    nsl = d // _CW
    ng = _CW // _L
    per_tile = E // _NS
    nchunk = per_tile // _K

    mesh = plsc.VectorSubcoreMesh(core_axis_name="c", subcore_axis_name="s")

    eij_t = ([jax.ShapeDtypeStruct((E, nsl, _CW), jnp.float32)]
             if write_eij else [])

    @functools.partial(
        pl.kernel,
        out_type=eij_t + [
            jax.ShapeDtypeStruct((N, nsl, _CW), jnp.float32),
            jax.ShapeDtypeStruct((N, nsl, _CW), jnp.float32),
        ],
        mesh=mesh,
        scratch_types=[
            pltpu.VMEM_SHARED((_NPAD, 1, _CW), jnp.float32),
            pltpu.VMEM((_K,), jnp.int32),
            pltpu.VMEM((_K,), jnp.int32),
            pltpu.VMEM((_K,), jnp.int32),
            pltpu.VMEM((_K,), jnp.int32),
            pltpu.VMEM((_K, 1, _CW), jnp.float32),
            pltpu.VMEM((_K, 1, _CW), jnp.float32),
            pltpu.VMEM((_K, 1, _CW), jnp.float32),
            pltpu.VMEM((_K, 1, _CW), jnp.float32),
            pltpu.VMEM((_K, 1, _CW), jnp.float32),
            pltpu.VMEM((_K, 1, _CW), jnp.float32),
            pltpu.VMEM((_K, 1, _CW), jnp.float32),
            pltpu.VMEM((_K, 1, _CW), jnp.float32),
            pltpu.SemaphoreType.DMA,
            pltpu.SemaphoreType.DMA,
        ],
    )
    def kern(bh, dh, eh, ce, src, dst, *out_and_scratch):
        if write_eij:
            eij_o, num_o, den_o = out_and_scratch[:3]
            (acc, sx0, dx0, sx1, dx1, bd0, be0, bc0, py0, bd1, be1, bc1, py1,
             sm0, sm1) = out_and_scratch[3:]
        else:
            eij_o = None
            num_o, den_o = out_and_scratch[:2]
            (acc, sx0, dx0, sx1, dx1, bd0, be0, bc0, py0, bd1, be1, bc1, py1,
             sm0, sm1) = out_and_scratch[2:]
        cid = jax.lax.axis_index("c")
        sid = jax.lax.axis_index("s")
        sets = ((sx0, dx0, bd0, be0, bc0, py0, sm0),
                (sx1, dx1, bd1, be1, bc1, py1, sm1))

        for p in range(nsl):
            j = cid * nsl + p
            is_num = (j % 2) == 0
            gq = j // 2
            if p > 0:
                plsc.subcore_barrier()

            def zb_row(r, _):
                for jg in range(ng):
                    py0[r, 0, pl.ds(jg * _L, _L)] = jnp.zeros((_L,), jnp.float32)
                return 0
            jax.lax.fori_loop(0, _K, zb_row, 0)
            for jz in range(_STRIPE // _K):
                pltpu.sync_copy(
                    py0, acc.at[pl.ds(sid * _STRIPE + jz * _K, _K)])
            plsc.subcore_barrier()

            def gath(tbl, idxrow, buf, sem):
                return pltpu.make_async_copy(tbl.at[idxrow, pl.ds(gq, 1)],
                                             buf, sem)

            def issue(i, s):
                sx, dx, bd, be, bc, py, sem = sets[s]
                base = sid * per_tile + i * _K
                pltpu.sync_copy(src.at[pl.ds(base, _K)], sx)
                pltpu.sync_copy(dst.at[pl.ds(base, _K)], dx)
                gath(dh, sx, bd, sem).start()
                gath(eh, dx, be, sem).start()
                pltpu.make_async_copy(ce.at[pl.ds(base, _K), pl.ds(gq, 1)],
                                      bc, sem).start()

                @pl.when(is_num)
                def _():
                    gath(bh, sx, py, sem).start()

            def finish(i, s):
                sx, dx, bd, be, bc, py, sem = sets[s]
                base = sid * per_tile + i * _K
                gath(dh, sx, bd, sem).wait()
                gath(eh, dx, be, sem).wait()
                pltpu.make_async_copy(ce.at[pl.ds(base, _K), pl.ds(gq, 1)],
                                      bc, sem).wait()

                @pl.when(is_num)
                def _():
                    gath(bh, sx, py, sem).wait()

                @pl.when(is_num)
                def _():
                    def rows_n(r2, _):
                        for u in range(2):
                            r = r2 * 2 + u
                            for jg in range(ng):
                                sl = pl.ds(jg * _L, _L)
                                eij = (bc[r, 0, sl] + bd[r, 0, sl]
                                       + be[r, 0, sl])
                                sg = 1.0 / (1.0 + jnp.exp(-eij))
                                sg = jnp.minimum(jnp.maximum(sg, 1e-4),
                                                 1.0 - 1e-4)
                                if write_eij:
                                    bc[r, 0, sl] = eij
                                py[r, 0, sl] = sg * py[r, 0, sl]
                        return 0
                    jax.lax.fori_loop(0, _K // 2, rows_n, 0)

                @pl.when(jnp.logical_not(is_num))
                def _():
                    def rows_d(r2, _):
                        for u in range(2):
                            r = r2 * 2 + u
                            for jg in range(ng):
                                sl = pl.ds(jg * _L, _L)
                                eij = (bc[r, 0, sl] + bd[r, 0, sl]
                                       + be[r, 0, sl])
                                sg = 1.0 / (1.0 + jnp.exp(-eij))
                                sg = jnp.minimum(jnp.maximum(sg, 1e-4),
                                                 1.0 - 1e-4)
                                py[r, 0, sl] = sg
                        return 0
                    jax.lax.fori_loop(0, _K // 2, rows_d, 0)

                if write_eij:
                    @pl.when(is_num)
                    def _():
                        pltpu.sync_copy(bc,
                                        eij_o.at[pl.ds(base, _K), pl.ds(gq, 1)])

                pltpu.sync_copy(py, acc.at[dx], add=True)

            issue(0, 0)

            def pair(ii, _):
                i0 = ii * 2
                issue(i0 + 1, 1)
                finish(i0, 0)

                @pl.when(i0 + 2 < nchunk)
                def _():
                    issue(i0 + 2, 0)

                finish(i0 + 1, 1)
                return 0
            jax.lax.fori_loop(0, nchunk // 2, pair, 0)

            plsc.subcore_barrier()
            last_valid = (N - (_NS - 1) * _STRIPE) // _K
            for jj in range(_STRIPE // _K):
                r0 = sid * _STRIPE + jj * _K
                @pl.when(jnp.logical_or(sid < _NS - 1, jj < last_valid))
                def _():
                    @pl.when(is_num)
                    def _():
                        pltpu.sync_copy(acc.at[pl.ds(r0, _K)],
                                        num_o.at[pl.ds(r0, _K), pl.ds(gq, 1)])

                    @pl.when(jnp.logical_not(is_num))
                    def _():
                        pltpu.sync_copy(acc.at[pl.ds(r0, _K)],
                                        den_o.at[pl.ds(r0, _K), pl.ds(gq, 1)])

    return kern


def _edge_phase(bh3, dh3, eh3, ce3, ei, write_eij=True):
    d = bh3.shape[1] * _CW
    kern = _edge_kernel(d, write_eij)
    return kern(bh3, dh3, eh3, ce3, ei[0], ei[1])


def _bn_gb(st, m, g, b, eps=1e-5):
    mean = st[0] / m
    var = st[1] / m - mean * mean
    gs = g.reshape(mean.shape) / jnp.sqrt(var + eps)
    bs = b.reshape(mean.shape) - mean * gs
    return jnp.concatenate(
        [gs[None], bs[None], jnp.zeros((6,) + mean.shape, jnp.float32)], axis=0)


def _w3(w):
    k, n = w.shape
    return w.reshape(k // _CW, _CW, n)


def _gated_layer(h3, e3, ei, p, snorm_n, snorm_e, ones_n, last=False):
    d = h3.shape[1] * _CW
    wcat = _w3(jnp.concatenate(
        [p['A_w'].T, p['B_w'].T, p['D_w'].T, p['E_w'].T], axis=1))
    bcat = jnp.concatenate([p['A_b'], p['B_b'], p['D_b'], p['E_b']])
    ah3, bh3, dh3, eh3 = _mm_node(h3, wcat, bcat, d)
    ce3 = _mm3(e3, _w3(p['C_w'].T), p['C_b'])
    if last:
        num3, den3 = _edge_phase(bh3, dh3, eh3, ce3, ei, write_eij=False)
    else:
        eij3, num3, den3 = _edge_phase(bh3, dh3, eh3, ce3, ei)
    hn3, hst = _prep_h(ah3, num3, den3, h3, snorm_n)
    gb_h = _bn_gb(hst, N, p['bn_h_g'], p['bn_h_b'])
    h_out = _apply(hn3, ones_n, h3, gb_h)
    if last:
        return h_out, None
    est = _stats_rows(eij3, snorm_e)
    gb_e = _bn_gb(est, E, p['bn_e_g'], p['bn_e_b'])
    e_out = _apply(eij3, snorm_e, e3, gb_e)
    return h_out, e_out


def kernel(feats, e_w, snorm_n, snorm_e, edge_index, maps, params):
    ei = edge_index
    ones_n = jnp.ones((N, 1), jnp.float32)

    h = _mm3(feats.reshape(N, HID // _CW, _CW), _w3(params['emb_h_w'].T),
             params['emb_h_b'])
    e = _outer(e_w, params['emb_e_w'][:, 0], params['emb_e_b'])

    h, e = _gated_layer(h, e, ei, params['inp1'], snorm_n, snorm_e, ones_n)
    h, e_inp = _gated_layer(h, e, ei, params['inp2'], snorm_n, snorm_e, ones_n)

    z = jax.random.normal(jax.random.key(42), (N, ZD), dtype=jnp.float32)
    h_dec = jnp.concatenate([h, z[:, None, :]], axis=1)
    e_dec = _mm3(e_inp, _w3(params['emb_edec_w'].T), params['emb_edec_b'])

    h2, e2 = _gated_layer(h_dec, e_dec, ei, params['dec1'], snorm_n,
                          snorm_e, ones_n)
    h2, _ = _gated_layer(h2, e2, ei, params['dec2'], snorm_n,
                         snorm_e, ones_n, last=True)

    x = _mm3(h2, _w3(params['mlp0_w'].T), params['mlp0_b'], relu=True,
             out3=False)
    x = _mm(x, params['mlp1_w'].T, params['mlp1_b'], relu=True)
    return _mm(x, params['mlp2_w'].T, params['mlp2_b'])


# speedup vs baseline: 1.5601x; 1.1173x over previous
import functools

import jax
import jax.numpy as jnp
from jax.experimental import pallas as pl
from jax.experimental.pallas import tpu as pltpu
from jax.experimental.pallas import tpu_sc as plsc

N = 10000
E = 160000
HID = 256
ZD = 128
DEC = HID + ZD

_NS = 16
_L = 16
_K = 40
_NPAD = 10240
_STRIPE = _NPAD // _NS


def _mm(x, w, b, relu=False):
    def body(x_ref, w_ref, b_ref, o_ref):
        acc = jnp.dot(x_ref[...], w_ref[...], preferred_element_type=jnp.float32)
        acc = acc + b_ref[0, :][None, :]
        if relu:
            acc = jnp.maximum(acc, 0.0)
        o_ref[...] = acc

    m, k = x.shape
    n = w.shape[1]
    bm = 640 if m == E else 400
    b8 = jnp.broadcast_to(b[None, :], (8, n))
    return pl.pallas_call(
        body,
        grid=(m // bm,),
        in_specs=[
            pl.BlockSpec((bm, k), lambda i: (i, 0)),
            pl.BlockSpec((k, n), lambda i: (0, 0)),
            pl.BlockSpec((8, n), lambda i: (0, 0)),
        ],
        out_specs=pl.BlockSpec((bm, n), lambda i: (i, 0)),
        out_shape=jax.ShapeDtypeStruct((m, n), jnp.float32),
    )(x, w, b8)


def _mm3(x3, w3, b, relu=False, out3=True):
    m, ki, _ = x3.shape
    n = w3.shape[2]
    no = n // _CW
    bm = 640 if m == E else 400
    b8 = jnp.broadcast_to(b[None, :], (8, n))

    def body(x_ref, w_ref, b_ref, o_ref):
        acc = jnp.dot(x_ref[:, 0, :], w_ref[0],
                      preferred_element_type=jnp.float32)
        for s in range(1, ki):
            acc += jnp.dot(x_ref[:, s, :], w_ref[s],
                           preferred_element_type=jnp.float32)
        acc = acc + b_ref[0, :][None, :]
        if relu:
            acc = jnp.maximum(acc, 0.0)
        if out3:
            for t in range(no):
                o_ref[:, t, :] = acc[:, t * _CW:(t + 1) * _CW]
        else:
            o_ref[...] = acc

    if out3:
        out_spec = pl.BlockSpec((bm, no, _CW), lambda i: (i, 0, 0))
        out_shape = jax.ShapeDtypeStruct((m, no, _CW), jnp.float32)
    else:
        out_spec = pl.BlockSpec((bm, n), lambda i: (i, 0))
        out_shape = jax.ShapeDtypeStruct((m, n), jnp.float32)
    return pl.pallas_call(
        body,
        grid=(m // bm,),
        in_specs=[
            pl.BlockSpec((bm, ki, _CW), lambda i: (i, 0, 0)),
            pl.BlockSpec((ki, _CW, n), lambda i: (0, 0, 0)),
            pl.BlockSpec((8, n), lambda i: (0, 0)),
        ],
        out_specs=out_spec,
        out_shape=out_shape,
    )(x3, w3, b8)


def _mm_node(h3, w3, b, d):
    m, ki, _ = h3.shape
    no = d // _CW
    bm = 400
    b8 = jnp.broadcast_to(b[None, :], (8, 4 * d))

    def body(x_ref, w_ref, b_ref, oa, ob, od, oe):
        acc = jnp.dot(x_ref[:, 0, :], w_ref[0],
                      preferred_element_type=jnp.float32)
        for s in range(1, ki):
            acc += jnp.dot(x_ref[:, s, :], w_ref[s],
                           preferred_element_type=jnp.float32)
        acc = acc + b_ref[0, :][None, :]
        for q, o_ref in enumerate((oa, ob, od, oe)):
            for t in range(no):
                c0 = q * d + t * _CW
                o_ref[:, t, :] = acc[:, c0:c0 + _CW]

    spec = pl.BlockSpec((bm, no, _CW), lambda i: (i, 0, 0))
    shp = jax.ShapeDtypeStruct((m, no, _CW), jnp.float32)
    return pl.pallas_call(
        body,
        grid=(m // bm,),
        in_specs=[
            pl.BlockSpec((bm, ki, _CW), lambda i: (i, 0, 0)),
            pl.BlockSpec((ki, _CW, 4 * d), lambda i: (0, 0, 0)),
            pl.BlockSpec((8, 4 * d), lambda i: (0, 0)),
        ],
        out_specs=[spec, spec, spec, spec],
        out_shape=[shp, shp, shp, shp],
    )(h3, w3, b8)


def _outer(e_w, w_col, b):
    hdim = w_col.shape[0]
    no = hdim // _CW
    wb = jnp.concatenate([w_col[None, :], b[None, :]], axis=0)
    wb8 = jnp.concatenate([wb, jnp.zeros((6, hdim), jnp.float32)], axis=0)
    bm = 640

    def body(ew_ref, wb_ref, o_ref):
        v = ew_ref[...] * wb_ref[0, :][None, :] + wb_ref[1, :][None, :]
        for t in range(no):
            o_ref[:, t, :] = v[:, t * _CW:(t + 1) * _CW]

    return pl.pallas_call(
        body,
        grid=(E // bm,),
        in_specs=[
            pl.BlockSpec((bm, 1), lambda i: (i, 0)),
            pl.BlockSpec((8, hdim), lambda i: (0, 0)),
        ],
        out_specs=pl.BlockSpec((bm, no, _CW), lambda i: (i, 0, 0)),
        out_shape=jax.ShapeDtypeStruct((E, no, _CW), jnp.float32),
    )(e_w, wb8)


def _preph_body(ah_ref, num_ref, den_ref, h_ref, sn_ref, hn_ref, st_ref):
    i = pl.program_id(0)
    den = den_ref[...]
    safe = jnp.where(den == 0.0, 1.0, den)
    hagg = ah_ref[...] + num_ref[...] / safe
    mask = den[:, :1, :1] > 0.0
    hnew = jnp.where(mask, hagg, h_ref[...])
    hp = hnew * sn_ref[...][:, :, None]
    hn_ref[...] = hp
    s = jnp.sum(hp, axis=0)
    s2 = jnp.sum(hp * hp, axis=0)
    upd = jnp.concatenate(
        [s[None], s2[None],
         jnp.zeros((6,) + s.shape, jnp.float32)], axis=0)

    @pl.when(i == 0)
    def _():
        st_ref[...] = jnp.zeros_like(st_ref)

    st_ref[...] += upd


def _prep_h(ah, num, den, h, snorm_n):
    m, nsl, _ = ah.shape
    bm = 400
    spec = pl.BlockSpec((bm, nsl, _CW), lambda i: (i, 0, 0))
    return pl.pallas_call(
        _preph_body,
        grid=(m // bm,),
        in_specs=[spec, spec, spec, spec,
                  pl.BlockSpec((bm, 1), lambda i: (i, 0))],
        out_specs=[spec, pl.BlockSpec((8, nsl, _CW), lambda i: (0, 0, 0))],
        out_shape=[
            jax.ShapeDtypeStruct((m, nsl, _CW), jnp.float32),
            jax.ShapeDtypeStruct((8, nsl, _CW), jnp.float32),
        ],
    )(ah, num, den, h, snorm_n)


def _statse_body(x_ref, rs_ref, st_ref):
    i = pl.program_id(0)
    y = x_ref[...] * rs_ref[...][:, :, None]
    s = jnp.sum(y, axis=0)
    s2 = jnp.sum(y * y, axis=0)
    upd = jnp.concatenate(
        [s[None], s2[None],
         jnp.zeros((6,) + s.shape, jnp.float32)], axis=0)

    @pl.when(i == 0)
    def _():
        st_ref[...] = jnp.zeros_like(st_ref)

    st_ref[...] += upd


def _stats_rows(x, rowscale):
    m, nsl, _ = x.shape
    bm = 640 if m == E else 400
    spec = pl.BlockSpec((bm, nsl, _CW), lambda i: (i, 0, 0))
    return pl.pallas_call(
        _statse_body,
        grid=(m // bm,),
        in_specs=[spec, pl.BlockSpec((bm, 1), lambda i: (i, 0))],
        out_specs=pl.BlockSpec((8, nsl, _CW), lambda i: (0, 0, 0)),
        out_shape=jax.ShapeDtypeStruct((8, nsl, _CW), jnp.float32),
    )(x, rowscale)


def _apply_body(x_ref, rs_ref, res_ref, gb_ref, o_ref):
    y = x_ref[...] * rs_ref[...][:, :, None]
    y = y * gb_ref[0][None] + gb_ref[1][None]
    o_ref[...] = res_ref[...] + jnp.maximum(y, 0.0)


def _apply(x, rowscale, res, gb):
    m, nsl, _ = x.shape
    bm = 640 if m == E else 400
    spec = pl.BlockSpec((bm, nsl, _CW), lambda i: (i, 0, 0))
    return pl.pallas_call(
        _apply_body,
        grid=(m // bm,),
        in_specs=[spec, pl.BlockSpec((bm, 1), lambda i: (i, 0)),
                  spec, pl.BlockSpec((8, nsl, _CW), lambda i: (0, 0, 0))],
        out_specs=spec,
        out_shape=jax.ShapeDtypeStruct((m, nsl, _CW), jnp.float32),
    )(x, rowscale, res, gb)


_CW = 128


@functools.lru_cache(maxsize=None)
def _edge_kernel(d, write_eij=True):
    nsl = d // _CW
    ng = _CW // _L
    per_tile = E // _NS
    nchunk = per_tile // _K

    mesh = plsc.VectorSubcoreMesh(core_axis_name="c", subcore_axis_name="s")

    eij_t = ([jax.ShapeDtypeStruct((E, nsl, _CW), jnp.float32)]
             if write_eij else [])

    @functools.partial(
        pl.kernel,
        out_type=eij_t + [
            jax.ShapeDtypeStruct((N, nsl, _CW), jnp.float32),
            jax.ShapeDtypeStruct((N, nsl, _CW), jnp.float32),
        ],
        mesh=mesh,
        scratch_types=[
            pltpu.VMEM_SHARED((_NPAD, 1, _CW), jnp.float32),
            pltpu.VMEM((_K,), jnp.int32),
            pltpu.VMEM((_K,), jnp.int32),
            pltpu.VMEM((_K,), jnp.int32),
            pltpu.VMEM((_K,), jnp.int32),
            pltpu.VMEM((_K, 1, _CW), jnp.float32),
            pltpu.VMEM((_K, 1, _CW), jnp.float32),
            pltpu.VMEM((_K, 1, _CW), jnp.float32),
            pltpu.VMEM((_K, 1, _CW), jnp.float32),
            pltpu.VMEM((_K, 1, _CW), jnp.float32),
            pltpu.VMEM((_K, 1, _CW), jnp.float32),
            pltpu.VMEM((_K, 1, _CW), jnp.float32),
            pltpu.VMEM((_K, 1, _CW), jnp.float32),
            pltpu.SemaphoreType.DMA,
            pltpu.SemaphoreType.DMA,
            pltpu.SemaphoreType.DMA,
            pltpu.SemaphoreType.DMA,
        ],
    )
    def kern(bh, dh, eh, ce, src, dst, *out_and_scratch):
        if write_eij:
            eij_o, num_o, den_o = out_and_scratch[:3]
            (acc, sx0, dx0, sx1, dx1, bd0, be0, bc0, py0, bd1, be1, bc1, py1,
             sm0, sm1, smi0, smi1) = out_and_scratch[3:]
        else:
            eij_o = None
            num_o, den_o = out_and_scratch[:2]
            (acc, sx0, dx0, sx1, dx1, bd0, be0, bc0, py0, bd1, be1, bc1, py1,
             sm0, sm1, smi0, smi1) = out_and_scratch[2:]
        cid = jax.lax.axis_index("c")
        sid = jax.lax.axis_index("s")
        sets = ((sx0, dx0, bd0, be0, bc0, py0, sm0, smi0),
                (sx1, dx1, bd1, be1, bc1, py1, sm1, smi1))

        for p in range(nsl):
            j = cid * nsl + p
            is_num = (j % 2) == 0
            gq = j // 2
            if p > 0:
                plsc.subcore_barrier()

            def zb_row(r, _):
                for jg in range(ng):
                    py0[r, 0, pl.ds(jg * _L, _L)] = jnp.zeros((_L,), jnp.float32)
                return 0
            jax.lax.fori_loop(0, _K, zb_row, 0)
            for jz in range(_STRIPE // _K):
                pltpu.sync_copy(
                    py0, acc.at[pl.ds(sid * _STRIPE + jz * _K, _K)])
            plsc.subcore_barrier()

            def gath(tbl, idxrow, buf, sem):
                return pltpu.make_async_copy(tbl.at[idxrow, pl.ds(gq, 1)],
                                             buf, sem)

            def sx_copy(i, s):
                sx = sets[s][0]
                smi = sets[s][7]
                base = sid * per_tile + i * _K
                return pltpu.make_async_copy(src.at[pl.ds(base, _K)], sx, smi)

            def issue_idx(i, s):
                sx_copy(i, s).start()

            def issue(i, s):
                sx, dx, bd, be, bc, py, sem, smi = sets[s]
                base = sid * per_tile + i * _K
                sx_copy(i, s).wait()
                pltpu.sync_copy(dst.at[pl.ds(base, _K)], dx)
                gath(dh, sx, bd, sem).start()
                gath(eh, dx, be, sem).start()
                pltpu.make_async_copy(ce.at[pl.ds(base, _K), pl.ds(gq, 1)],
                                      bc, sem).start()

                @pl.when(is_num)
                def _():
                    gath(bh, sx, py, sem).start()

            def finish(i, s):
                sx, dx, bd, be, bc, py, sem, smi = sets[s]
                base = sid * per_tile + i * _K
                gath(dh, sx, bd, sem).wait()
                gath(eh, dx, be, sem).wait()
                pltpu.make_async_copy(ce.at[pl.ds(base, _K), pl.ds(gq, 1)],
                                      bc, sem).wait()

                @pl.when(is_num)
                def _():
                    gath(bh, sx, py, sem).wait()

                @pl.when(i + 2 < nchunk)
                def _():
                    issue_idx(i + 2, s)

                @pl.when(is_num)
                def _():
                    def rows_n(r2, _):
                        for u in range(2):
                            r = r2 * 2 + u
                            for jg in range(ng):
                                sl = pl.ds(jg * _L, _L)
                                eij = (bc[r, 0, sl] + bd[r, 0, sl]
                                       + be[r, 0, sl])
                                sg = 1.0 / (1.0 + jnp.exp(-eij))
                                sg = jnp.minimum(jnp.maximum(sg, 1e-4),
                                                 1.0 - 1e-4)
                                if write_eij:
                                    bc[r, 0, sl] = eij
                                py[r, 0, sl] = sg * py[r, 0, sl]
                        return 0
                    jax.lax.fori_loop(0, _K // 2, rows_n, 0)

                @pl.when(jnp.logical_not(is_num))
                def _():
                    def rows_d(r2, _):
                        for u in range(2):
                            r = r2 * 2 + u
                            for jg in range(ng):
                                sl = pl.ds(jg * _L, _L)
                                eij = (bc[r, 0, sl] + bd[r, 0, sl]
                                       + be[r, 0, sl])
                                sg = 1.0 / (1.0 + jnp.exp(-eij))
                                sg = jnp.minimum(jnp.maximum(sg, 1e-4),
                                                 1.0 - 1e-4)
                                py[r, 0, sl] = sg
                        return 0
                    jax.lax.fori_loop(0, _K // 2, rows_d, 0)

                if write_eij:
                    @pl.when(is_num)
                    def _():
                        pltpu.sync_copy(bc,
                                        eij_o.at[pl.ds(base, _K), pl.ds(gq, 1)])

                pltpu.sync_copy(py, acc.at[dx], add=True)

            issue_idx(0, 0)
            issue_idx(1, 1)
            issue(0, 0)

            def pair(ii, _):
                i0 = ii * 2
                issue(i0 + 1, 1)
                finish(i0, 0)

                @pl.when(i0 + 2 < nchunk)
                def _():
                    issue(i0 + 2, 0)

                finish(i0 + 1, 1)
                return 0
            jax.lax.fori_loop(0, nchunk // 2, pair, 0)

            plsc.subcore_barrier()
            last_valid = (N - (_NS - 1) * _STRIPE) // _K
            for jj in range(_STRIPE // _K):
                r0 = sid * _STRIPE + jj * _K
                @pl.when(jnp.logical_or(sid < _NS - 1, jj < last_valid))
                def _():
                    @pl.when(is_num)
                    def _():
                        pltpu.sync_copy(acc.at[pl.ds(r0, _K)],
                                        num_o.at[pl.ds(r0, _K), pl.ds(gq, 1)])

                    @pl.when(jnp.logical_not(is_num))
                    def _():
                        pltpu.sync_copy(acc.at[pl.ds(r0, _K)],
                                        den_o.at[pl.ds(r0, _K), pl.ds(gq, 1)])

    return kern


def _edge_phase(bh3, dh3, eh3, ce3, ei, write_eij=True):
    d = bh3.shape[1] * _CW
    kern = _edge_kernel(d, write_eij)
    return kern(bh3, dh3, eh3, ce3, ei[0], ei[1])


def _bn_gb(st, m, g, b, eps=1e-5):
    mean = st[0] / m
    var = st[1] / m - mean * mean
    gs = g.reshape(mean.shape) / jnp.sqrt(var + eps)
    bs = b.reshape(mean.shape) - mean * gs
    return jnp.concatenate(
        [gs[None], bs[None], jnp.zeros((6,) + mean.shape, jnp.float32)], axis=0)


def _w3(w):
    k, n = w.shape
    return w.reshape(k // _CW, _CW, n)


def _gated_layer(h3, e3, ei, p, snorm_n, snorm_e, ones_n, last=False):
    d = h3.shape[1] * _CW
    wcat = _w3(jnp.concatenate(
        [p['A_w'].T, p['B_w'].T, p['D_w'].T, p['E_w'].T], axis=1))
    bcat = jnp.concatenate([p['A_b'], p['B_b'], p['D_b'], p['E_b']])
    ah3, bh3, dh3, eh3 = _mm_node(h3, wcat, bcat, d)
    ce3 = _mm3(e3, _w3(p['C_w'].T), p['C_b'])
    if last:
        num3, den3 = _edge_phase(bh3, dh3, eh3, ce3, ei, write_eij=False)
    else:
        eij3, num3, den3 = _edge_phase(bh3, dh3, eh3, ce3, ei)
    hn3, hst = _prep_h(ah3, num3, den3, h3, snorm_n)
    gb_h = _bn_gb(hst, N, p['bn_h_g'], p['bn_h_b'])
    h_out = _apply(hn3, ones_n, h3, gb_h)
    if last:
        return h_out, None
    est = _stats_rows(eij3, snorm_e)
    gb_e = _bn_gb(est, E, p['bn_e_g'], p['bn_e_b'])
    e_out = _apply(eij3, snorm_e, e3, gb_e)
    return h_out, e_out


def kernel(feats, e_w, snorm_n, snorm_e, edge_index, maps, params):
    ei = edge_index
    ones_n = jnp.ones((N, 1), jnp.float32)

    h = _mm3(feats.reshape(N, HID // _CW, _CW), _w3(params['emb_h_w'].T),
             params['emb_h_b'])
    e = _outer(e_w, params['emb_e_w'][:, 0], params['emb_e_b'])

    h, e = _gated_layer(h, e, ei, params['inp1'], snorm_n, snorm_e, ones_n)
    h, e_inp = _gated_layer(h, e, ei, params['inp2'], snorm_n, snorm_e, ones_n)

    z = jax.random.normal(jax.random.key(42), (N, ZD), dtype=jnp.float32)
    h_dec = jnp.concatenate([h, z[:, None, :]], axis=1)
    e_dec = _mm3(e_inp, _w3(params['emb_edec_w'].T), params['emb_edec_b'])

    h2, e2 = _gated_layer(h_dec, e_dec, ei, params['dec1'], snorm_n,
                          snorm_e, ones_n)
    h2, _ = _gated_layer(h2, e2, ei, params['dec2'], snorm_n,
                         snorm_e, ones_n, last=True)

    x = _mm3(h2, _w3(params['mlp0_w'].T), params['mlp0_b'], relu=True,
             out3=False)
    x = _mm(x, params['mlp1_w'].T, params['mlp1_b'], relu=True)
    return _mm(x, params['mlp2_w'].T, params['mlp2_b'])


# speedup vs baseline: 1.5615x; 1.0009x over previous
import functools

import jax
import jax.numpy as jnp
from jax.experimental import pallas as pl
from jax.experimental.pallas import tpu as pltpu
from jax.experimental.pallas import tpu_sc as plsc

N = 10000
E = 160000
HID = 256
ZD = 128
DEC = HID + ZD

_NS = 16
_L = 16
_K = 40
_NPAD = 10240
_STRIPE = _NPAD // _NS


def _mm(x, w, b, relu=False):
    def body(x_ref, w_ref, b_ref, o_ref):
        acc = jnp.dot(x_ref[...], w_ref[...], preferred_element_type=jnp.float32)
        acc = acc + b_ref[0, :][None, :]
        if relu:
            acc = jnp.maximum(acc, 0.0)
        o_ref[...] = acc

    m, k = x.shape
    n = w.shape[1]
    bm = 640 if m == E else 400
    b8 = jnp.broadcast_to(b[None, :], (8, n))
    return pl.pallas_call(
        body,
        grid=(m // bm,),
        in_specs=[
            pl.BlockSpec((bm, k), lambda i: (i, 0)),
            pl.BlockSpec((k, n), lambda i: (0, 0)),
            pl.BlockSpec((8, n), lambda i: (0, 0)),
        ],
        out_specs=pl.BlockSpec((bm, n), lambda i: (i, 0)),
        out_shape=jax.ShapeDtypeStruct((m, n), jnp.float32),
    )(x, w, b8)


def _mm3(x3, w3, b, relu=False, out3=True):
    m, ki, _ = x3.shape
    n = w3.shape[2]
    no = n // _CW
    bm = 640 if m == E else 400
    b8 = jnp.broadcast_to(b[None, :], (8, n))

    def body(x_ref, w_ref, b_ref, o_ref):
        acc = jnp.dot(x_ref[:, 0, :], w_ref[0],
                      preferred_element_type=jnp.float32)
        for s in range(1, ki):
            acc += jnp.dot(x_ref[:, s, :], w_ref[s],
                           preferred_element_type=jnp.float32)
        acc = acc + b_ref[0, :][None, :]
        if relu:
            acc = jnp.maximum(acc, 0.0)
        if out3:
            for t in range(no):
                o_ref[:, t, :] = acc[:, t * _CW:(t + 1) * _CW]
        else:
            o_ref[...] = acc

    if out3:
        out_spec = pl.BlockSpec((bm, no, _CW), lambda i: (i, 0, 0))
        out_shape = jax.ShapeDtypeStruct((m, no, _CW), jnp.float32)
    else:
        out_spec = pl.BlockSpec((bm, n), lambda i: (i, 0))
        out_shape = jax.ShapeDtypeStruct((m, n), jnp.float32)
    return pl.pallas_call(
        body,
        grid=(m // bm,),
        in_specs=[
            pl.BlockSpec((bm, ki, _CW), lambda i: (i, 0, 0)),
            pl.BlockSpec((ki, _CW, n), lambda i: (0, 0, 0)),
            pl.BlockSpec((8, n), lambda i: (0, 0)),
        ],
        out_specs=out_spec,
        out_shape=out_shape,
    )(x3, w3, b8)


def _mm_node(h3, w3, b, d):
    m, ki, _ = h3.shape
    no = d // _CW
    bm = 400
    b8 = jnp.broadcast_to(b[None, :], (8, 4 * d))

    def body(x_ref, w_ref, b_ref, oa, ob, od, oe):
        acc = jnp.dot(x_ref[:, 0, :], w_ref[0],
                      preferred_element_type=jnp.float32)
        for s in range(1, ki):
            acc += jnp.dot(x_ref[:, s, :], w_ref[s],
                           preferred_element_type=jnp.float32)
        acc = acc + b_ref[0, :][None, :]
        for q, o_ref in enumerate((oa, ob, od, oe)):
            for t in range(no):
                c0 = q * d + t * _CW
                o_ref[:, t, :] = acc[:, c0:c0 + _CW]

    spec = pl.BlockSpec((bm, no, _CW), lambda i: (i, 0, 0))
    shp = jax.ShapeDtypeStruct((m, no, _CW), jnp.float32)
    return pl.pallas_call(
        body,
        grid=(m // bm,),
        in_specs=[
            pl.BlockSpec((bm, ki, _CW), lambda i: (i, 0, 0)),
            pl.BlockSpec((ki, _CW, 4 * d), lambda i: (0, 0, 0)),
            pl.BlockSpec((8, 4 * d), lambda i: (0, 0)),
        ],
        out_specs=[spec, spec, spec, spec],
        out_shape=[shp, shp, shp, shp],
    )(h3, w3, b8)


def _outer(e_w, w_col, b):
    hdim = w_col.shape[0]
    no = hdim // _CW
    wb = jnp.concatenate([w_col[None, :], b[None, :]], axis=0)
    wb8 = jnp.concatenate([wb, jnp.zeros((6, hdim), jnp.float32)], axis=0)
    bm = 640

    def body(ew_ref, wb_ref, o_ref):
        v = ew_ref[...] * wb_ref[0, :][None, :] + wb_ref[1, :][None, :]
        for t in range(no):
            o_ref[:, t, :] = v[:, t * _CW:(t + 1) * _CW]

    return pl.pallas_call(
        body,
        grid=(E // bm,),
        in_specs=[
            pl.BlockSpec((bm, 1), lambda i: (i, 0)),
            pl.BlockSpec((8, hdim), lambda i: (0, 0)),
        ],
        out_specs=pl.BlockSpec((bm, no, _CW), lambda i: (i, 0, 0)),
        out_shape=jax.ShapeDtypeStruct((E, no, _CW), jnp.float32),
    )(e_w, wb8)


def _preph_body(ah_ref, num_ref, den_ref, h_ref, sn_ref, hn_ref, st_ref):
    i = pl.program_id(0)
    den = den_ref[...]
    safe = jnp.where(den == 0.0, 1.0, den)
    hagg = ah_ref[...] + num_ref[...] / safe
    mask = den[:, :1, :1] > 0.0
    hnew = jnp.where(mask, hagg, h_ref[...])
    hp = hnew * sn_ref[...][:, :, None]
    hn_ref[...] = hp
    s = jnp.sum(hp, axis=0)
    s2 = jnp.sum(hp * hp, axis=0)
    upd = jnp.concatenate(
        [s[None], s2[None],
         jnp.zeros((6,) + s.shape, jnp.float32)], axis=0)

    @pl.when(i == 0)
    def _():
        st_ref[...] = jnp.zeros_like(st_ref)

    st_ref[...] += upd


def _prep_h(ah, num, den, h, snorm_n):
    m, nsl, _ = ah.shape
    bm = 400
    spec = pl.BlockSpec((bm, nsl, _CW), lambda i: (i, 0, 0))
    return pl.pallas_call(
        _preph_body,
        grid=(m // bm,),
        in_specs=[spec, spec, spec, spec,
                  pl.BlockSpec((bm, 1), lambda i: (i, 0))],
        out_specs=[spec, pl.BlockSpec((8, nsl, _CW), lambda i: (0, 0, 0))],
        out_shape=[
            jax.ShapeDtypeStruct((m, nsl, _CW), jnp.float32),
            jax.ShapeDtypeStruct((8, nsl, _CW), jnp.float32),
        ],
    )(ah, num, den, h, snorm_n)


def _statse_body(x_ref, rs_ref, st_ref):
    i = pl.program_id(0)
    y = x_ref[...] * rs_ref[...][:, :, None]
    s = jnp.sum(y, axis=0)
    s2 = jnp.sum(y * y, axis=0)
    upd = jnp.concatenate(
        [s[None], s2[None],
         jnp.zeros((6,) + s.shape, jnp.float32)], axis=0)

    @pl.when(i == 0)
    def _():
        st_ref[...] = jnp.zeros_like(st_ref)

    st_ref[...] += upd


def _stats_rows(x, rowscale):
    m, nsl, _ = x.shape
    bm = 640 if m == E else 400
    spec = pl.BlockSpec((bm, nsl, _CW), lambda i: (i, 0, 0))
    return pl.pallas_call(
        _statse_body,
        grid=(m // bm,),
        in_specs=[spec, pl.BlockSpec((bm, 1), lambda i: (i, 0))],
        out_specs=pl.BlockSpec((8, nsl, _CW), lambda i: (0, 0, 0)),
        out_shape=jax.ShapeDtypeStruct((8, nsl, _CW), jnp.float32),
    )(x, rowscale)


def _apply_body(x_ref, rs_ref, res_ref, gb_ref, o_ref):
    y = x_ref[...] * rs_ref[...][:, :, None]
    y = y * gb_ref[0][None] + gb_ref[1][None]
    o_ref[...] = res_ref[...] + jnp.maximum(y, 0.0)


def _apply(x, rowscale, res, gb):
    m, nsl, _ = x.shape
    bm = 640 if m == E else 400
    spec = pl.BlockSpec((bm, nsl, _CW), lambda i: (i, 0, 0))
    return pl.pallas_call(
        _apply_body,
        grid=(m // bm,),
        in_specs=[spec, pl.BlockSpec((bm, 1), lambda i: (i, 0)),
                  spec, pl.BlockSpec((8, nsl, _CW), lambda i: (0, 0, 0))],
        out_specs=spec,
        out_shape=jax.ShapeDtypeStruct((m, nsl, _CW), jnp.float32),
    )(x, rowscale, res, gb)


_CW = 128


@functools.lru_cache(maxsize=None)
def _edge_kernel(d, write_eij=True):
    nsl = d // _CW
    ng = _CW // _L
    per_tile = E // _NS
    nchunk = per_tile // _K

    mesh = plsc.VectorSubcoreMesh(core_axis_name="c", subcore_axis_name="s")

    eij_t = ([jax.ShapeDtypeStruct((E, nsl, _CW), jnp.float32)]
             if write_eij else [])

    @functools.partial(
        pl.kernel,
        out_type=eij_t + [
            jax.ShapeDtypeStruct((N, nsl, _CW), jnp.float32),
            jax.ShapeDtypeStruct((N, nsl, _CW), jnp.float32),
        ],
        mesh=mesh,
        scratch_types=[
            pltpu.VMEM_SHARED((_NPAD, 1, _CW), jnp.float32),
            pltpu.VMEM((_K,), jnp.int32),
            pltpu.VMEM((_K,), jnp.int32),
            pltpu.VMEM((_K,), jnp.int32),
            pltpu.VMEM((_K,), jnp.int32),
            pltpu.VMEM((_K, 1, _CW), jnp.float32),
            pltpu.VMEM((_K, 1, _CW), jnp.float32),
            pltpu.VMEM((_K, 1, _CW), jnp.float32),
            pltpu.VMEM((_K, 1, _CW), jnp.float32),
            pltpu.VMEM((_K, 1, _CW), jnp.float32),
            pltpu.VMEM((_K, 1, _CW), jnp.float32),
            pltpu.VMEM((_K, 1, _CW), jnp.float32),
            pltpu.VMEM((_K, 1, _CW), jnp.float32),
        ] + [pltpu.SemaphoreType.DMA] * 4,
    )
    def kern(bh, dh, eh, ce, src, dst, *out_and_scratch):
        if write_eij:
            eij_o, num_o, den_o = out_and_scratch[:3]
            scr = out_and_scratch[3:]
        else:
            eij_o = None
            num_o, den_o = out_and_scratch[:2]
            scr = out_and_scratch[2:]
        (acc, sx0, dx0, sx1, dx1, bd0, be0, bc0, py0, bd1, be1, bc1, py1,
         sm0, sm1, smi0, smi1) = scr
        cid = jax.lax.axis_index("c")
        sid = jax.lax.axis_index("s")
        sets = ((sx0, dx0, bd0, be0, bc0, py0, sm0, smi0),
                (sx1, dx1, bd1, be1, bc1, py1, sm1, smi1))

        for p in range(nsl):
            j = cid * nsl + p
            is_num = (j % 2) == 0
            gq = j // 2
            if p > 0:
                plsc.subcore_barrier()

            def zb_row(r, _):
                for jg in range(ng):
                    py0[r, 0, pl.ds(jg * _L, _L)] = jnp.zeros((_L,), jnp.float32)
                return 0
            jax.lax.fori_loop(0, _K, zb_row, 0)
            for jz in range(_STRIPE // _K):
                pltpu.sync_copy(
                    py0, acc.at[pl.ds(sid * _STRIPE + jz * _K, _K)])
            plsc.subcore_barrier()

            def gath(tbl, idxrow, buf, sem):
                return pltpu.make_async_copy(tbl.at[idxrow, pl.ds(gq, 1)],
                                             buf, sem)

            def sx_copy(i, s):
                sx = sets[s][0]
                smi = sets[s][7]
                base = sid * per_tile + i * _K
                return pltpu.make_async_copy(src.at[pl.ds(base, _K)], sx, smi)

            def issue(i, s):
                sx, dx, bd, be, bc, py, sem, smi = sets[s]
                base = sid * per_tile + i * _K
                sx_copy(i, s).wait()
                pltpu.sync_copy(dst.at[pl.ds(base, _K)], dx)
                gath(dh, sx, bd, sem).start()
                gath(eh, dx, be, sem).start()
                pltpu.make_async_copy(ce.at[pl.ds(base, _K), pl.ds(gq, 1)],
                                      bc, sem).start()

                @pl.when(is_num)
                def _():
                    gath(bh, sx, py, sem).start()

            def finish(i, s):
                sx, dx, bd, be, bc, py, sem, smi = sets[s]
                base = sid * per_tile + i * _K
                gath(dh, sx, bd, sem).wait()
                gath(eh, dx, be, sem).wait()
                pltpu.make_async_copy(ce.at[pl.ds(base, _K), pl.ds(gq, 1)],
                                      bc, sem).wait()

                @pl.when(is_num)
                def _():
                    gath(bh, sx, py, sem).wait()

                @pl.when(i + 2 < nchunk)
                def _():
                    sx_copy(i + 2, s).start()

                @pl.when(is_num)
                def _():
                    def rows_n(r2, _):
                        for u in range(2):
                            r = r2 * 2 + u
                            for jg in range(ng):
                                sl = pl.ds(jg * _L, _L)
                                eij = (bc[r, 0, sl] + bd[r, 0, sl]
                                       + be[r, 0, sl])
                                sg = 1.0 / (1.0 + jnp.exp(-eij))
                                sg = jnp.minimum(jnp.maximum(sg, 1e-4),
                                                 1.0 - 1e-4)
                                if write_eij:
                                    bc[r, 0, sl] = eij
                                py[r, 0, sl] = sg * py[r, 0, sl]
                        return 0
                    jax.lax.fori_loop(0, _K // 2, rows_n, 0)

                @pl.when(jnp.logical_not(is_num))
                def _():
                    def rows_d(r2, _):
                        for u in range(2):
                            r = r2 * 2 + u
                            for jg in range(ng):
                                sl = pl.ds(jg * _L, _L)
                                eij = (bc[r, 0, sl] + bd[r, 0, sl]
                                       + be[r, 0, sl])
                                sg = 1.0 / (1.0 + jnp.exp(-eij))
                                sg = jnp.minimum(jnp.maximum(sg, 1e-4),
                                                 1.0 - 1e-4)
                                py[r, 0, sl] = sg
                        return 0
                    jax.lax.fori_loop(0, _K // 2, rows_d, 0)

                if write_eij:
                    @pl.when(is_num)
                    def _():
                        pltpu.sync_copy(bc,
                                        eij_o.at[pl.ds(base, _K), pl.ds(gq, 1)])

                pltpu.sync_copy(py, acc.at[dx], add=True)

            sx_copy(0, 0).start()
            sx_copy(1, 1).start()
            issue(0, 0)

            def pair(ii, _):
                i0 = ii * 2
                issue(i0 + 1, 1)
                finish(i0, 0)

                @pl.when(i0 + 2 < nchunk)
                def _():
                    issue(i0 + 2, 0)

                finish(i0 + 1, 1)
                return 0
            jax.lax.fori_loop(0, nchunk // 2, pair, 0)

            plsc.subcore_barrier()
            last_valid = (N - (_NS - 1) * _STRIPE) // _K
            for jj in range(_STRIPE // _K):
                r0 = sid * _STRIPE + jj * _K
                @pl.when(jnp.logical_or(sid < _NS - 1, jj < last_valid))
                def _():
                    @pl.when(is_num)
                    def _():
                        pltpu.sync_copy(acc.at[pl.ds(r0, _K)],
                                        num_o.at[pl.ds(r0, _K), pl.ds(gq, 1)])

                    @pl.when(jnp.logical_not(is_num))
                    def _():
                        pltpu.sync_copy(acc.at[pl.ds(r0, _K)],
                                        den_o.at[pl.ds(r0, _K), pl.ds(gq, 1)])

    return kern


def _edge_phase(bh3, dh3, eh3, ce3, ei, write_eij=True):
    d = bh3.shape[1] * _CW
    kern = _edge_kernel(d, write_eij)
    return kern(bh3, dh3, eh3, ce3, ei[0], ei[1])


def _bn_gb(st, m, g, b, eps=1e-5):
    mean = st[0] / m
    var = st[1] / m - mean * mean
    gs = g.reshape(mean.shape) / jnp.sqrt(var + eps)
    bs = b.reshape(mean.shape) - mean * gs
    return jnp.concatenate(
        [gs[None], bs[None], jnp.zeros((6,) + mean.shape, jnp.float32)], axis=0)


def _w3(w):
    k, n = w.shape
    return w.reshape(k // _CW, _CW, n)


def _gated_layer(h3, e3, ei, p, snorm_n, snorm_e, ones_n, last=False):
    d = h3.shape[1] * _CW
    wcat = _w3(jnp.concatenate(
        [p['A_w'].T, p['B_w'].T, p['D_w'].T, p['E_w'].T], axis=1))
    bcat = jnp.concatenate([p['A_b'], p['B_b'], p['D_b'], p['E_b']])
    ah3, bh3, dh3, eh3 = _mm_node(h3, wcat, bcat, d)
    ce3 = _mm3(e3, _w3(p['C_w'].T), p['C_b'])
    if last:
        num3, den3 = _edge_phase(bh3, dh3, eh3, ce3, ei, write_eij=False)
    else:
        eij3, num3, den3 = _edge_phase(bh3, dh3, eh3, ce3, ei)
    hn3, hst = _prep_h(ah3, num3, den3, h3, snorm_n)
    gb_h = _bn_gb(hst, N, p['bn_h_g'], p['bn_h_b'])
    h_out = _apply(hn3, ones_n, h3, gb_h)
    if last:
        return h_out, None
    est = _stats_rows(eij3, snorm_e)
    gb_e = _bn_gb(est, E, p['bn_e_g'], p['bn_e_b'])
    e_out = _apply(eij3, snorm_e, e3, gb_e)
    return h_out, e_out


def kernel(feats, e_w, snorm_n, snorm_e, edge_index, maps, params):
    ei = edge_index
    ones_n = jnp.ones((N, 1), jnp.float32)

    h = _mm3(feats.reshape(N, HID // _CW, _CW), _w3(params['emb_h_w'].T),
             params['emb_h_b'])
    e = _outer(e_w, params['emb_e_w'][:, 0], params['emb_e_b'])

    h, e = _gated_layer(h, e, ei, params['inp1'], snorm_n, snorm_e, ones_n)
    h, e_inp = _gated_layer(h, e, ei, params['inp2'], snorm_n, snorm_e, ones_n)

    z = jax.random.normal(jax.random.key(42), (N, ZD), dtype=jnp.float32)
    h_dec = jnp.concatenate([h, z[:, None, :]], axis=1)
    e_dec = _mm3(e_inp, _w3(params['emb_edec_w'].T), params['emb_edec_b'])

    h2, e2 = _gated_layer(h_dec, e_dec, ei, params['dec1'], snorm_n,
                          snorm_e, ones_n)
    h2, _ = _gated_layer(h2, e2, ei, params['dec2'], snorm_n,
                         snorm_e, ones_n, last=True)

    x = _mm3(h2, _w3(params['mlp0_w'].T), params['mlp0_b'], relu=True,
             out3=False)
    x = _mm(x, params['mlp1_w'].T, params['mlp1_b'], relu=True)
    return _mm(x, params['mlp2_w'].T, params['mlp2_b'])
